# baseline (speedup 1.0000x reference)
import jax
import jax.numpy as jnp
from jax import lax
from jax.experimental import pallas as pl
from jax.experimental.pallas import tpu as pltpu

_N = 16384
_B = 8
_D = 192
_DH = 96
_VQ = 64
_NE = 1024
_NK = _N // _B
_HALF = _NK // 2


def _vq_pack_body(js_ref, y0_ref, y1_ref, wenc_ref, wpre_ref, cb_ref,
                  se_ref, so_ref, te_ref, to_ref, pi_ref, po_ref):
    y0 = y0_ref[...]
    y1 = y1_ref[...]
    w0 = wenc_ref[0:_DH, :]
    w1 = wenc_ref[_DH:_D, :]
    dn = (((1,), (0,)), ((), ()))
    h = jnp.maximum(
        lax.dot_general(y0, w0, dn, precision=lax.Precision.DEFAULT)
        + lax.dot_general(y1, w1, dn, precision=lax.Precision.DEFAULT),
        0.0)
    e = lax.dot_general(h, wpre_ref[...], dn, precision=lax.Precision.DEFAULT)

    cb = cb_ref[...]
    cbn = jnp.sum(cb * cb, axis=1)[None, :]
    ii = lax.broadcasted_iota(jnp.int32, (_NK, _NE), 1)

    def _level(r, need_residual):
        rn = jnp.sum(r * r, axis=1, keepdims=True)
        sc = lax.dot_general(r, cb, (((1,), (1,)), ((), ())),
                             precision=lax.Precision.DEFAULT)
        dists = rn - 2.0 * sc + cbn
        m = jnp.min(dists, axis=1, keepdims=True)
        idx = jnp.min(jnp.where(dists == m, ii, _NE), axis=1)
        if not need_residual:
            return idx, None
        oh = (ii == idx[:, None]).astype(jnp.float32)
        c = lax.dot_general(oh, cb, dn, precision=lax.Precision.HIGHEST)
        return idx, r - c

    idx0, r1 = _level(e, True)
    idx1, _ = _level(r1, False)

    j = js_ref[pl.program_id(0)]
    A = idx0[None, :]
    Bv = idx1[None, :]
    A0 = A[:, 0:_HALF]
    A1 = A[:, 1:_HALF + 1]
    A2 = A[:, 2:_HALF + 2]
    B0 = Bv[:, 0:_HALF]
    B1 = Bv[:, 1:_HALF + 1]
    Bm1 = jnp.concatenate(
        [jnp.full((1, 1), -3, jnp.int32), Bv[:, 0:_HALF - 1]], axis=1)

    def sel4(v0, v1, v2, v3):
        return jnp.where(j == 0, v0,
               jnp.where(j == 1, v1,
               jnp.where(j == 2, v2, v3)))

    se_ref[...] = (sel4(Bm1, A0, B0, A1) + 3)[None]
    so_ref[...] = (sel4(A0, B0, A1, B1) + 3)[None]
    te_ref[...] = (sel4(A0, B0, A1, B1) + 3)[None]
    to_ref[...] = (sel4(B0, A1, B1, A2) + 3)[None]

    t = lax.broadcasted_iota(jnp.int32, (1, _NK), 1)
    p = t + j
    pi = jnp.where(p == 0, 0, ((p - 1) & 1) + 3)
    shift = jnp.maximum(j - 1, 0) // 2
    po = jnp.where(p == 0, 0, ((p - 1) >> 1) + 3 - shift)
    pi_ref[...] = pi[None]
    po_ref[...] = po[None]


def _vq_pack(js, y2, W_enc, W_pre, codebook):
    i32 = jnp.int32
    out_shapes = (
        jax.ShapeDtypeStruct((_B, 1, _HALF), i32),
        jax.ShapeDtypeStruct((_B, 1, _HALF), i32),
        jax.ShapeDtypeStruct((_B, 1, _HALF), i32),
        jax.ShapeDtypeStruct((_B, 1, _HALF), i32),
        jax.ShapeDtypeStruct((_B, 1, _NK), i32),
        jax.ShapeDtypeStruct((_B, 1, _NK), i32),
    )
    grid = (_B,)
    in_specs = [
        pl.BlockSpec((_B,), lambda k: (0,), memory_space=pltpu.SMEM),
        pl.BlockSpec((_NK, _DH), lambda k: (k, 0)),
        pl.BlockSpec((_NK, _DH), lambda k: (k + _B, 0)),
        pl.BlockSpec((_D, _D), lambda k: (0, 0)),
        pl.BlockSpec((_D, _VQ), lambda k: (0, 0)),
        pl.BlockSpec((_NE, _VQ), lambda k: (0, 0)),
    ]
    out_specs = (
        pl.BlockSpec((1, 1, _HALF), lambda k: (k, 0, 0)),
        pl.BlockSpec((1, 1, _HALF), lambda k: (k, 0, 0)),
        pl.BlockSpec((1, 1, _HALF), lambda k: (k, 0, 0)),
        pl.BlockSpec((1, 1, _HALF), lambda k: (k, 0, 0)),
        pl.BlockSpec((1, 1, _NK), lambda k: (k, 0, 0)),
        pl.BlockSpec((1, 1, _NK), lambda k: (k, 0, 0)),
    )
    return pl.pallas_call(
        _vq_pack_body,
        grid=grid,
        in_specs=in_specs,
        out_specs=out_specs,
        out_shape=out_shapes,
    )(js, y2, y2, W_enc, W_pre, codebook)


def kernel(x, edge_index, batch, faces, num_vertices, js, W_enc, W_pre, codebook):
    del batch, faces, num_vertices
    src = edge_index[0].astype(jnp.int32)
    dst = edge_index[1].astype(jnp.int32)
    js = js.astype(jnp.int32)

    agg = jax.ops.segment_sum(jnp.take(x, src, axis=0), dst, num_segments=_N)
    y = x + agg
    y2 = jnp.concatenate([y[:, :_DH], y[:, _DH:]], axis=0)

    se, so, te, to, pi, po = _vq_pack(js, y2, W_enc, W_pre, codebook)

    idxt = jax.dtypes.canonicalize_dtype(jnp.int64)
    S = jnp.stack([se.reshape(_B, _HALF), so.reshape(_B, _HALF)],
                  axis=-1).reshape(_B, _NK).astype(idxt)
    T = jnp.stack([te.reshape(_B, _HALF), to.reshape(_B, _HALF)],
                  axis=-1).reshape(_B, _NK).astype(idxt)
    PI = pi.reshape(_B, _NK).astype(idxt)
    PO = po.reshape(_B, _NK).astype(idxt)
    return S, T, PI, PO


# speedup vs baseline: 3.1643x; 3.1643x over previous
import functools

import jax
import jax.numpy as jnp
from jax import lax
from jax.experimental import pallas as pl
from jax.experimental.pallas import tpu as pltpu
from jax.experimental.pallas import tpu_sc as plsc

_N = 16384
_B = 8
_D = 192
_DH = 96
_VQ = 64
_NE = 1024
_NK = _N // _B
_HALF = _NK // 2

_E = 131072
_NSC = 16
_EPT = _E // _NSC
_CH = 128
_NCH = _EPT // _CH
_GRP = 4
_NGRP = _NCH // _GRP
_RPT = _N // _NSC


_DQ = _D // 4


def _segsum_body(src_hbm, dst_hbm, x4_hbm, out_hbm, srcv, dstv, rows, sem, shared):
    c = lax.axis_index("c")
    s = lax.axis_index("s")
    row0 = s * _RPT

    for t in range(2):
        q = 2 * c + t
        base = q * _N + row0
        for i in range(_RPT // _CH):
            pltpu.sync_copy(x4_hbm.at[pl.ds(base + i * _CH, _CH)],
                            shared.at[pl.ds(row0 + i * _CH, _CH)])
        pltpu.sync_copy(src_hbm.at[q, pl.ds(s * _NCH, _NCH)], srcv)
        if t == 0:
            pltpu.sync_copy(dst_hbm.at[pl.ds(s * _NCH, _NCH)], dstv)
        plsc.subcore_barrier()

        def group(g, carry):
            handles = []
            for b in range(_GRP):
                ch = g * _GRP + b
                handles.append(
                    pltpu.async_copy(x4_hbm.at[srcv.at[ch]], rows.at[b], sem))
            for h in handles:
                h.wait()
            for b in range(_GRP):
                ch = g * _GRP + b
                pltpu.sync_copy(rows.at[b], shared.at[dstv.at[ch]], add=True)
            return carry

        lax.fori_loop(0, _NGRP, group, 0)
        plsc.subcore_barrier()

        for i in range(_RPT // _CH):
            pltpu.sync_copy(shared.at[pl.ds(row0 + i * _CH, _CH)],
                            out_hbm.at[pl.ds(base + i * _CH, _CH)])
        plsc.subcore_barrier()


@functools.partial(
    pl.kernel,
    out_type=jax.ShapeDtypeStruct((4 * _N, _DQ), jnp.float32),
    mesh=plsc.VectorSubcoreMesh(core_axis_name="c", subcore_axis_name="s"),
    compiler_params=pltpu.CompilerParams(use_tc_tiling_on_sc=False),
    scratch_types=[
        pltpu.VMEM((_NCH, _CH), jnp.int32),
        pltpu.VMEM((_NCH, _CH), jnp.int32),
        pltpu.VMEM((_GRP, _CH, _DQ), jnp.float32),
        pltpu.SemaphoreType.DMA,
        pltpu.VMEM_SHARED((_N, _DQ), jnp.float32),
    ],
)
def _segsum_sc(src_hbm, dst_hbm, x4_hbm, out_hbm, srcv, dstv, rows, sem, shared):
    _segsum_body(src_hbm, dst_hbm, x4_hbm, out_hbm, srcv, dstv, rows, sem, shared)


def _vq_pack_body(js_ref, y0_ref, y1_ref, y2_ref, y3_ref, wenc_ref, wpre_ref,
                  cb_ref, se_ref, so_ref, te_ref, to_ref, pi_ref, po_ref):
    dn = (((1,), (0,)), ((), ()))
    acc = None
    for q, yq_ref in enumerate((y0_ref, y1_ref, y2_ref, y3_ref)):
        wq = wenc_ref[q * _DQ:(q + 1) * _DQ, :]
        part = lax.dot_general(yq_ref[...], wq, dn,
                               precision=lax.Precision.DEFAULT)
        acc = part if acc is None else acc + part
    h = jnp.maximum(acc, 0.0)
    e = lax.dot_general(h, wpre_ref[...], dn, precision=lax.Precision.DEFAULT)

    cb = cb_ref[...]
    cbn = jnp.sum(cb * cb, axis=1)[None, :]
    ii = lax.broadcasted_iota(jnp.int32, (_NK, _NE), 1)

    def _level(r, need_residual):
        rn = jnp.sum(r * r, axis=1, keepdims=True)
        sc = lax.dot_general(r, cb, (((1,), (1,)), ((), ())),
                             precision=lax.Precision.DEFAULT)
        dists = rn - 2.0 * sc + cbn
        m = jnp.min(dists, axis=1, keepdims=True)
        idx = jnp.min(jnp.where(dists == m, ii, _NE), axis=1)
        if not need_residual:
            return idx, None
        oh = (ii == idx[:, None]).astype(jnp.float32)
        c = lax.dot_general(oh, cb, dn, precision=lax.Precision.HIGHEST)
        return idx, r - c

    idx0, r1 = _level(e, True)
    idx1, _ = _level(r1, False)

    j = js_ref[pl.program_id(0)]
    A = idx0[None, :]
    Bv = idx1[None, :]
    A0 = A[:, 0:_HALF]
    A1 = A[:, 1:_HALF + 1]
    A2 = A[:, 2:_HALF + 2]
    B0 = Bv[:, 0:_HALF]
    B1 = Bv[:, 1:_HALF + 1]
    Bm1 = jnp.concatenate(
        [jnp.full((1, 1), -3, jnp.int32), Bv[:, 0:_HALF - 1]], axis=1)

    def sel4(v0, v1, v2, v3):
        return jnp.where(j == 0, v0,
               jnp.where(j == 1, v1,
               jnp.where(j == 2, v2, v3)))

    se_ref[...] = (sel4(Bm1, A0, B0, A1) + 3)[None]
    so_ref[...] = (sel4(A0, B0, A1, B1) + 3)[None]
    te_ref[...] = (sel4(A0, B0, A1, B1) + 3)[None]
    to_ref[...] = (sel4(B0, A1, B1, A2) + 3)[None]

    t = lax.broadcasted_iota(jnp.int32, (1, _NK), 1)
    p = t + j
    pi = jnp.where(p == 0, 0, ((p - 1) & 1) + 3)
    shift = jnp.maximum(j - 1, 0) // 2
    po = jnp.where(p == 0, 0, ((p - 1) >> 1) + 3 - shift)
    pi_ref[...] = pi[None]
    po_ref[...] = po[None]


def _vq_pack(js, y4, W_enc, W_pre, codebook):
    i32 = jnp.int32
    out_shapes = (
        jax.ShapeDtypeStruct((_B, 1, _HALF), i32),
        jax.ShapeDtypeStruct((_B, 1, _HALF), i32),
        jax.ShapeDtypeStruct((_B, 1, _HALF), i32),
        jax.ShapeDtypeStruct((_B, 1, _HALF), i32),
        jax.ShapeDtypeStruct((_B, 1, _NK), i32),
        jax.ShapeDtypeStruct((_B, 1, _NK), i32),
    )
    grid = (_B,)
    in_specs = [
        pl.BlockSpec((_B,), lambda k: (0,), memory_space=pltpu.SMEM),
        pl.BlockSpec((_NK, _DQ), lambda k: (k, 0)),
        pl.BlockSpec((_NK, _DQ), lambda k: (k + _B, 0)),
        pl.BlockSpec((_NK, _DQ), lambda k: (k + 2 * _B, 0)),
        pl.BlockSpec((_NK, _DQ), lambda k: (k + 3 * _B, 0)),
        pl.BlockSpec((_D, _D), lambda k: (0, 0)),
        pl.BlockSpec((_D, _VQ), lambda k: (0, 0)),
        pl.BlockSpec((_NE, _VQ), lambda k: (0, 0)),
    ]
    out_specs = (
        pl.BlockSpec((1, 1, _HALF), lambda k: (k, 0, 0)),
        pl.BlockSpec((1, 1, _HALF), lambda k: (k, 0, 0)),
        pl.BlockSpec((1, 1, _HALF), lambda k: (k, 0, 0)),
        pl.BlockSpec((1, 1, _HALF), lambda k: (k, 0, 0)),
        pl.BlockSpec((1, 1, _NK), lambda k: (k, 0, 0)),
        pl.BlockSpec((1, 1, _NK), lambda k: (k, 0, 0)),
    )
    return pl.pallas_call(
        _vq_pack_body,
        grid=grid,
        in_specs=in_specs,
        out_specs=out_specs,
        out_shape=out_shapes,
    )(js, y4, y4, y4, y4, W_enc, W_pre, codebook)


def kernel(x, edge_index, batch, faces, num_vertices, js, W_enc, W_pre, codebook):
    del batch, faces, num_vertices
    src = edge_index[0].astype(jnp.int32)
    dst = edge_index[1].astype(jnp.int32)
    js = js.astype(jnp.int32)

    x4 = jnp.concatenate([x[:, q * _DQ:(q + 1) * _DQ] for q in range(4)],
                         axis=0)
    src4 = jnp.stack([src + q * _N for q in range(4)]).reshape(
        4, _E // _CH, _CH)
    dst2 = dst.reshape(_E // _CH, _CH)
    y4 = _segsum_sc(src4, dst2, x4)

    se, so, te, to, pi, po = _vq_pack(js, y4, W_enc, W_pre, codebook)

    idxt = jax.dtypes.canonicalize_dtype(jnp.int64)
    S = jnp.stack([se.reshape(_B, _HALF), so.reshape(_B, _HALF)],
                  axis=-1).reshape(_B, _NK).astype(idxt)
    T = jnp.stack([te.reshape(_B, _HALF), to.reshape(_B, _HALF)],
                  axis=-1).reshape(_B, _NK).astype(idxt)
    PI = pi.reshape(_B, _NK).astype(idxt)
    PO = po.reshape(_B, _NK).astype(idxt)
    return S, T, PI, PO


# speedup vs baseline: 3.4082x; 1.0771x over previous
import functools

import jax
import jax.numpy as jnp
from jax import lax
from jax.experimental import pallas as pl
from jax.experimental.pallas import tpu as pltpu
from jax.experimental.pallas import tpu_sc as plsc

_N = 16384
_B = 8
_D = 192
_DH = 96
_VQ = 64
_NE = 1024
_NK = _N // _B
_HALF = _NK // 2

_E = 131072
_NSC = 16
_EPT = _E // _NSC
_CH = 128
_NCH = _EPT // _CH
_GRP = 4
_NGRP = _NCH // _GRP
_RPT = _N // _NSC


_DQ = _D // 4


def _segsum_body(src_hbm, dst_hbm, x4_hbm, out_hbm, srcv, dstv, rows, sem, shared):
    c = lax.axis_index("c")
    s = lax.axis_index("s")
    row0 = s * _RPT

    for t in range(2):
        q = 2 * c + t
        base = q * _N + row0
        for i in range(_RPT // _CH):
            pltpu.sync_copy(x4_hbm.at[pl.ds(base + i * _CH, _CH)],
                            shared.at[pl.ds(row0 + i * _CH, _CH)])
        pltpu.sync_copy(src_hbm.at[q, pl.ds(s * _NCH, _NCH)], srcv)
        if t == 0:
            pltpu.sync_copy(dst_hbm.at[pl.ds(s * _NCH, _NCH)], dstv)
        plsc.subcore_barrier()

        def group(g, carry):
            handles = []
            for b in range(_GRP):
                ch = g * _GRP + b
                handles.append(
                    pltpu.async_copy(x4_hbm.at[srcv.at[ch]], rows.at[b], sem))
            for h in handles:
                h.wait()
            for b in range(_GRP):
                ch = g * _GRP + b
                pltpu.sync_copy(rows.at[b], shared.at[dstv.at[ch]], add=True)
            return carry

        lax.fori_loop(0, _NGRP, group, 0)
        plsc.subcore_barrier()

        for i in range(_RPT // _CH):
            pltpu.sync_copy(shared.at[pl.ds(row0 + i * _CH, _CH)],
                            out_hbm.at[pl.ds(row0 + i * _CH, _CH),
                                       pl.ds(q * _DQ, _DQ)])
        plsc.subcore_barrier()


@functools.partial(
    pl.kernel,
    out_type=jax.ShapeDtypeStruct((_N, _D), jnp.float32),
    mesh=plsc.VectorSubcoreMesh(core_axis_name="c", subcore_axis_name="s"),
    compiler_params=pltpu.CompilerParams(use_tc_tiling_on_sc=False),
    scratch_types=[
        pltpu.VMEM((_NCH, _CH), jnp.int32),
        pltpu.VMEM((_NCH, _CH), jnp.int32),
        pltpu.VMEM((_GRP, _CH, _DQ), jnp.float32),
        pltpu.SemaphoreType.DMA,
        pltpu.VMEM_SHARED((_N, _DQ), jnp.float32),
    ],
)
def _segsum_sc(src_hbm, dst_hbm, x4_hbm, out_hbm, srcv, dstv, rows, sem, shared):
    _segsum_body(src_hbm, dst_hbm, x4_hbm, out_hbm, srcv, dstv, rows, sem, shared)


def _vq_pack_body(js_ref, y_ref, wenc_ref, wpre_ref,
                  cb_ref, se_ref, so_ref, te_ref, to_ref, pi_ref, po_ref):
    dn = (((1,), (0,)), ((), ()))
    h = jnp.maximum(
        lax.dot_general(y_ref[...], wenc_ref[...], dn,
                        precision=lax.Precision.DEFAULT), 0.0)
    e = lax.dot_general(h, wpre_ref[...], dn, precision=lax.Precision.DEFAULT)

    cb = cb_ref[...]
    cbn = jnp.sum(cb * cb, axis=1)[None, :]
    ii = lax.broadcasted_iota(jnp.int32, (_NK, _NE), 1)

    def _level(r, need_residual):
        rn = jnp.sum(r * r, axis=1, keepdims=True)
        sc = lax.dot_general(r, cb, (((1,), (1,)), ((), ())),
                             precision=lax.Precision.DEFAULT)
        dists = rn - 2.0 * sc + cbn
        m = jnp.min(dists, axis=1, keepdims=True)
        idx = jnp.min(jnp.where(dists == m, ii, _NE), axis=1)
        if not need_residual:
            return idx, None
        oh = (ii == idx[:, None]).astype(jnp.float32)
        c = lax.dot_general(oh, cb, dn, precision=lax.Precision.HIGHEST)
        return idx, r - c

    idx0, r1 = _level(e, True)
    idx1, _ = _level(r1, False)

    j = js_ref[pl.program_id(0)]
    A = idx0[None, :]
    Bv = idx1[None, :]
    A0 = A[:, 0:_HALF]
    A1 = A[:, 1:_HALF + 1]
    A2 = A[:, 2:_HALF + 2]
    B0 = Bv[:, 0:_HALF]
    B1 = Bv[:, 1:_HALF + 1]
    Bm1 = jnp.concatenate(
        [jnp.full((1, 1), -3, jnp.int32), Bv[:, 0:_HALF - 1]], axis=1)

    def sel4(v0, v1, v2, v3):
        return jnp.where(j == 0, v0,
               jnp.where(j == 1, v1,
               jnp.where(j == 2, v2, v3)))

    se_ref[...] = (sel4(Bm1, A0, B0, A1) + 3)[None]
    so_ref[...] = (sel4(A0, B0, A1, B1) + 3)[None]
    te_ref[...] = (sel4(A0, B0, A1, B1) + 3)[None]
    to_ref[...] = (sel4(B0, A1, B1, A2) + 3)[None]

    t = lax.broadcasted_iota(jnp.int32, (1, _NK), 1)
    p = t + j
    pi = jnp.where(p == 0, 0, ((p - 1) & 1) + 3)
    shift = jnp.maximum(j - 1, 0) // 2
    po = jnp.where(p == 0, 0, ((p - 1) >> 1) + 3 - shift)
    pi_ref[...] = pi[None]
    po_ref[...] = po[None]


def _vq_pack(js, y, W_enc, W_pre, codebook):
    i32 = jnp.int32
    out_shapes = (
        jax.ShapeDtypeStruct((_B, 1, _HALF), i32),
        jax.ShapeDtypeStruct((_B, 1, _HALF), i32),
        jax.ShapeDtypeStruct((_B, 1, _HALF), i32),
        jax.ShapeDtypeStruct((_B, 1, _HALF), i32),
        jax.ShapeDtypeStruct((_B, 1, _NK), i32),
        jax.ShapeDtypeStruct((_B, 1, _NK), i32),
    )
    grid = (_B,)
    in_specs = [
        pl.BlockSpec((_B,), lambda k: (0,), memory_space=pltpu.SMEM),
        pl.BlockSpec((_NK, _D), lambda k: (k, 0)),
        pl.BlockSpec((_D, _D), lambda k: (0, 0)),
        pl.BlockSpec((_D, _VQ), lambda k: (0, 0)),
        pl.BlockSpec((_NE, _VQ), lambda k: (0, 0)),
    ]
    out_specs = (
        pl.BlockSpec((1, 1, _HALF), lambda k: (k, 0, 0)),
        pl.BlockSpec((1, 1, _HALF), lambda k: (k, 0, 0)),
        pl.BlockSpec((1, 1, _HALF), lambda k: (k, 0, 0)),
        pl.BlockSpec((1, 1, _HALF), lambda k: (k, 0, 0)),
        pl.BlockSpec((1, 1, _NK), lambda k: (k, 0, 0)),
        pl.BlockSpec((1, 1, _NK), lambda k: (k, 0, 0)),
    )
    return pl.pallas_call(
        _vq_pack_body,
        grid=grid,
        in_specs=in_specs,
        out_specs=out_specs,
        out_shape=out_shapes,
    )(js, y, W_enc, W_pre, codebook)


def kernel(x, edge_index, batch, faces, num_vertices, js, W_enc, W_pre, codebook):
    del batch, faces, num_vertices
    src = edge_index[0].astype(jnp.int32)
    dst = edge_index[1].astype(jnp.int32)
    js = js.astype(jnp.int32)

    x4 = jnp.concatenate([x[:, q * _DQ:(q + 1) * _DQ] for q in range(4)],
                         axis=0)
    src4 = jnp.stack([src + q * _N for q in range(4)]).reshape(
        4, _E // _CH, _CH)
    dst2 = dst.reshape(_E // _CH, _CH)
    y = _segsum_sc(src4, dst2, x4)

    se, so, te, to, pi, po = _vq_pack(js, y, W_enc, W_pre, codebook)

    idxt = jax.dtypes.canonicalize_dtype(jnp.int64)
    S = jnp.stack([se.reshape(_B, _HALF), so.reshape(_B, _HALF)],
                  axis=-1).reshape(_B, _NK).astype(idxt)
    T = jnp.stack([te.reshape(_B, _HALF), to.reshape(_B, _HALF)],
                  axis=-1).reshape(_B, _NK).astype(idxt)
    PI = pi.reshape(_B, _NK).astype(idxt)
    PO = po.reshape(_B, _NK).astype(idxt)
    return S, T, PI, PO


# speedup vs baseline: 3.4987x; 1.0265x over previous
import functools

import jax
import jax.numpy as jnp
from jax import lax
from jax.experimental import pallas as pl
from jax.experimental.pallas import tpu as pltpu
from jax.experimental.pallas import tpu_sc as plsc

_N = 16384
_B = 8
_D = 192
_DH = 96
_VQ = 64
_NE = 1024
_NK = _N // _B
_HALF = _NK // 2

_E = 131072
_NSC = 16
_EPT = _E // _NSC
_CH = 128
_NCH = _EPT // _CH
_GRP = 4
_NGRP = _NCH // _GRP
_RPT = _N // _NSC


_DQ = _D // 4


def _segsum_body(src_hbm, dst_hbm, x4_hbm, out_hbm, srcv, dstv, rows, sem,
                 semb, shared):
    c = lax.axis_index("c")
    s = lax.axis_index("s")
    row0 = s * _RPT

    for t in range(2):
        q = 2 * c + t
        base = q * _N + row0
        for i in range(_RPT // _CH):
            pltpu.sync_copy(x4_hbm.at[pl.ds(base + i * _CH, _CH)],
                            shared.at[pl.ds(row0 + i * _CH, _CH)])
        pltpu.sync_copy(src_hbm.at[q, pl.ds(s * _NCH, _NCH)], srcv)
        if t == 0:
            pltpu.sync_copy(dst_hbm.at[pl.ds(s * _NCH, _NCH)], dstv)
        plsc.subcore_barrier()

        def g_start(g, pbuf, sm):
            for b in range(_GRP):
                pltpu.make_async_copy(x4_hbm.at[srcv.at[g * _GRP + b]],
                                      rows.at[pbuf, b], sm).start()

        def g_wait(g, pbuf, sm):
            for b in range(_GRP):
                pltpu.make_async_copy(x4_hbm.at[srcv.at[g * _GRP + b]],
                                      rows.at[pbuf, b], sm).wait()

        def g_scatter(g, pbuf):
            for b in range(_GRP):
                pltpu.sync_copy(rows.at[pbuf, b],
                                shared.at[dstv.at[g * _GRP + b]], add=True)

        g_start(0, 0, sem)

        def pair(i, carry):
            g0 = 2 * i
            g_start(g0 + 1, 1, semb)
            g_wait(g0, 0, sem)
            g_scatter(g0, 0)

            @pl.when(i < _NGRP // 2 - 1)
            def _():
                g_start(g0 + 2, 0, sem)

            g_wait(g0 + 1, 1, semb)
            g_scatter(g0 + 1, 1)
            return carry

        lax.fori_loop(0, _NGRP // 2, pair, 0)
        plsc.subcore_barrier()

        for i in range(_RPT // _CH):
            pltpu.sync_copy(shared.at[pl.ds(row0 + i * _CH, _CH)],
                            out_hbm.at[pl.ds(base + i * _CH, _CH)])
        plsc.subcore_barrier()


@functools.partial(
    pl.kernel,
    out_type=jax.ShapeDtypeStruct((4 * _N, _DQ), jnp.float32),
    mesh=plsc.VectorSubcoreMesh(core_axis_name="c", subcore_axis_name="s"),
    compiler_params=pltpu.CompilerParams(use_tc_tiling_on_sc=False),
    scratch_types=[
        pltpu.VMEM((_NCH, _CH), jnp.int32),
        pltpu.VMEM((_NCH, _CH), jnp.int32),
        pltpu.VMEM((2, _GRP, _CH, _DQ), jnp.float32),
        pltpu.SemaphoreType.DMA,
        pltpu.SemaphoreType.DMA,
        pltpu.VMEM_SHARED((_N, _DQ), jnp.float32),
    ],
)
def _segsum_sc(src_hbm, dst_hbm, x4_hbm, out_hbm, srcv, dstv, rows, sem, semb,
               shared):
    _segsum_body(src_hbm, dst_hbm, x4_hbm, out_hbm, srcv, dstv, rows, sem,
                 semb, shared)


def _vq_pack_body(js_ref, y0_ref, y1_ref, y2_ref, y3_ref, wenc_ref, wpre_ref,
                  cb_ref, se_ref, so_ref, te_ref, to_ref, pi_ref, po_ref):
    dn = (((1,), (0,)), ((), ()))
    acc = None
    for q, yq_ref in enumerate((y0_ref, y1_ref, y2_ref, y3_ref)):
        wq = wenc_ref[q * _DQ:(q + 1) * _DQ, :]
        part = lax.dot_general(yq_ref[...], wq, dn,
                               precision=lax.Precision.DEFAULT)
        acc = part if acc is None else acc + part
    h = jnp.maximum(acc, 0.0)
    e = lax.dot_general(h, wpre_ref[...], dn, precision=lax.Precision.DEFAULT)

    cb = cb_ref[...]
    cbn = jnp.sum(cb * cb, axis=1)[None, :]
    ii = lax.broadcasted_iota(jnp.int32, (_NK, _NE), 1)

    def _level(r, need_residual):
        rn = jnp.sum(r * r, axis=1, keepdims=True)
        sc = lax.dot_general(r, cb, (((1,), (1,)), ((), ())),
                             precision=lax.Precision.DEFAULT)
        dists = rn - 2.0 * sc + cbn
        m = jnp.min(dists, axis=1, keepdims=True)
        idx = jnp.min(jnp.where(dists == m, ii, _NE), axis=1)
        if not need_residual:
            return idx, None
        oh = (ii == idx[:, None]).astype(jnp.float32)
        c = lax.dot_general(oh, cb, dn, precision=lax.Precision.HIGHEST)
        return idx, r - c

    idx0, r1 = _level(e, True)
    idx1, _ = _level(r1, False)

    j = js_ref[pl.program_id(0)]
    A = idx0[None, :]
    Bv = idx1[None, :]
    A0 = A[:, 0:_HALF]
    A1 = A[:, 1:_HALF + 1]
    A2 = A[:, 2:_HALF + 2]
    B0 = Bv[:, 0:_HALF]
    B1 = Bv[:, 1:_HALF + 1]
    Bm1 = jnp.concatenate(
        [jnp.full((1, 1), -3, jnp.int32), Bv[:, 0:_HALF - 1]], axis=1)

    def sel4(v0, v1, v2, v3):
        return jnp.where(j == 0, v0,
               jnp.where(j == 1, v1,
               jnp.where(j == 2, v2, v3)))

    se_ref[...] = (sel4(Bm1, A0, B0, A1) + 3)[None]
    so_ref[...] = (sel4(A0, B0, A1, B1) + 3)[None]
    te_ref[...] = (sel4(A0, B0, A1, B1) + 3)[None]
    to_ref[...] = (sel4(B0, A1, B1, A2) + 3)[None]

    t = lax.broadcasted_iota(jnp.int32, (1, _NK), 1)
    p = t + j
    pi = jnp.where(p == 0, 0, ((p - 1) & 1) + 3)
    shift = jnp.maximum(j - 1, 0) // 2
    po = jnp.where(p == 0, 0, ((p - 1) >> 1) + 3 - shift)
    pi_ref[...] = pi[None]
    po_ref[...] = po[None]


def _vq_pack(js, y4, W_enc, W_pre, codebook):
    i32 = jnp.int32
    out_shapes = (
        jax.ShapeDtypeStruct((_B, 1, _HALF), i32),
        jax.ShapeDtypeStruct((_B, 1, _HALF), i32),
        jax.ShapeDtypeStruct((_B, 1, _HALF), i32),
        jax.ShapeDtypeStruct((_B, 1, _HALF), i32),
        jax.ShapeDtypeStruct((_B, 1, _NK), i32),
        jax.ShapeDtypeStruct((_B, 1, _NK), i32),
    )
    grid = (_B,)
    in_specs = [
        pl.BlockSpec((_B,), lambda k: (0,), memory_space=pltpu.SMEM),
        pl.BlockSpec((_NK, _DQ), lambda k: (k, 0)),
        pl.BlockSpec((_NK, _DQ), lambda k: (k + _B, 0)),
        pl.BlockSpec((_NK, _DQ), lambda k: (k + 2 * _B, 0)),
        pl.BlockSpec((_NK, _DQ), lambda k: (k + 3 * _B, 0)),
        pl.BlockSpec((_D, _D), lambda k: (0, 0)),
        pl.BlockSpec((_D, _VQ), lambda k: (0, 0)),
        pl.BlockSpec((_NE, _VQ), lambda k: (0, 0)),
    ]
    out_specs = (
        pl.BlockSpec((1, 1, _HALF), lambda k: (k, 0, 0)),
        pl.BlockSpec((1, 1, _HALF), lambda k: (k, 0, 0)),
        pl.BlockSpec((1, 1, _HALF), lambda k: (k, 0, 0)),
        pl.BlockSpec((1, 1, _HALF), lambda k: (k, 0, 0)),
        pl.BlockSpec((1, 1, _NK), lambda k: (k, 0, 0)),
        pl.BlockSpec((1, 1, _NK), lambda k: (k, 0, 0)),
    )
    return pl.pallas_call(
        _vq_pack_body,
        grid=grid,
        in_specs=in_specs,
        out_specs=out_specs,
        out_shape=out_shapes,
    )(js, y4, y4, y4, y4, W_enc, W_pre, codebook)


def kernel(x, edge_index, batch, faces, num_vertices, js, W_enc, W_pre, codebook):
    del batch, faces, num_vertices
    src = edge_index[0].astype(jnp.int32)
    dst = edge_index[1].astype(jnp.int32)
    js = js.astype(jnp.int32)

    x4 = jnp.concatenate([x[:, q * _DQ:(q + 1) * _DQ] for q in range(4)],
                         axis=0)
    src4 = jnp.stack([src + q * _N for q in range(4)]).reshape(
        4, _E // _CH, _CH)
    dst2 = dst.reshape(_E // _CH, _CH)
    y4 = _segsum_sc(src4, dst2, x4)

    se, so, te, to, pi, po = _vq_pack(js, y4, W_enc, W_pre, codebook)

    idxt = jax.dtypes.canonicalize_dtype(jnp.int64)
    S = jnp.stack([se.reshape(_B, _HALF), so.reshape(_B, _HALF)],
                  axis=-1).reshape(_B, _NK).astype(idxt)
    T = jnp.stack([te.reshape(_B, _HALF), to.reshape(_B, _HALF)],
                  axis=-1).reshape(_B, _NK).astype(idxt)
    PI = pi.reshape(_B, _NK).astype(idxt)
    PO = po.reshape(_B, _NK).astype(idxt)
    return S, T, PI, PO


# speedup vs baseline: 3.8221x; 1.0924x over previous
import functools

import jax
import jax.numpy as jnp
from jax import lax
from jax.experimental import pallas as pl
from jax.experimental.pallas import tpu as pltpu
from jax.experimental.pallas import tpu_sc as plsc

_N = 16384
_B = 8
_D = 192
_DH = 96
_VQ = 64
_NE = 1024
_NK = _N // _B
_HALF = _NK // 2

_E = 131072
_NSC = 16
_EPT = _E // _NSC
_CH = 128
_NCH = _EPT // _CH
_GRP = 4
_NGRP = _NCH // _GRP
_RPT = _N // _NSC


_DQ = _D // 4


def _segsum_body(src_hbm, dst_hbm, x4_hbm, out_hbm, srcv, dstv, rows, sem,
                 semb, shared):
    c = lax.axis_index("c")
    s = lax.axis_index("s")
    row0 = s * _RPT

    for t in range(2):
        q = 2 * c + t
        base = q * _N + row0
        for i in range(_RPT // _CH):
            pltpu.sync_copy(x4_hbm.at[pl.ds(base + i * _CH, _CH)],
                            shared.at[pl.ds(row0 + i * _CH, _CH)])
        pltpu.sync_copy(src_hbm.at[q, pl.ds(s * _NCH, _NCH)], srcv)
        if t == 0:
            pltpu.sync_copy(dst_hbm.at[pl.ds(s * _NCH, _NCH)], dstv)
        plsc.subcore_barrier()

        def g_start(g, pbuf, sm):
            for b in range(_GRP):
                pltpu.make_async_copy(x4_hbm.at[srcv.at[g * _GRP + b]],
                                      rows.at[pbuf, b], sm).start()

        def g_wait(g, pbuf, sm):
            for b in range(_GRP):
                pltpu.make_async_copy(x4_hbm.at[srcv.at[g * _GRP + b]],
                                      rows.at[pbuf, b], sm).wait()

        def g_scatter(g, pbuf):
            for b in range(_GRP):
                pltpu.sync_copy(rows.at[pbuf, b],
                                shared.at[dstv.at[g * _GRP + b]], add=True)

        g_start(0, 0, sem)

        def pair(i, carry):
            g0 = 2 * i
            g_start(g0 + 1, 1, semb)
            g_wait(g0, 0, sem)
            g_scatter(g0, 0)

            @pl.when(i < _NGRP // 2 - 1)
            def _():
                g_start(g0 + 2, 0, sem)

            g_wait(g0 + 1, 1, semb)
            g_scatter(g0 + 1, 1)
            return carry

        lax.fori_loop(0, _NGRP // 2, pair, 0)
        plsc.subcore_barrier()

        for i in range(_RPT // _CH):
            pltpu.sync_copy(shared.at[pl.ds(row0 + i * _CH, _CH)],
                            out_hbm.at[pl.ds(base + i * _CH, _CH)])
        plsc.subcore_barrier()


@functools.partial(
    pl.kernel,
    out_type=jax.ShapeDtypeStruct((4 * _N, _DQ), jnp.float32),
    mesh=plsc.VectorSubcoreMesh(core_axis_name="c", subcore_axis_name="s"),
    compiler_params=pltpu.CompilerParams(use_tc_tiling_on_sc=False),
    scratch_types=[
        pltpu.VMEM((_NCH, _CH), jnp.int32),
        pltpu.VMEM((_NCH, _CH), jnp.int32),
        pltpu.VMEM((2, _GRP, _CH, _DQ), jnp.float32),
        pltpu.SemaphoreType.DMA,
        pltpu.SemaphoreType.DMA,
        pltpu.VMEM_SHARED((_N, _DQ), jnp.float32),
    ],
)
def _segsum_sc(src_hbm, dst_hbm, x4_hbm, out_hbm, srcv, dstv, rows, sem, semb,
               shared):
    _segsum_body(src_hbm, dst_hbm, x4_hbm, out_hbm, srcv, dstv, rows, sem,
                 semb, shared)


def _vq_pack_body(js_ref, y0_ref, y1_ref, y2_ref, y3_ref, wenc_ref, wpre_ref,
                  cb_ref, se_ref, so_ref, te_ref, to_ref, pi_ref, po_ref):
    dn = (((1,), (0,)), ((), ()))
    acc = None
    for q, yq_ref in enumerate((y0_ref, y1_ref, y2_ref, y3_ref)):
        wq = wenc_ref[q * _DQ:(q + 1) * _DQ, :]
        part = lax.dot_general(yq_ref[...], wq, dn,
                               precision=lax.Precision.DEFAULT)
        acc = part if acc is None else acc + part
    h = jnp.maximum(acc, 0.0)
    e = lax.dot_general(h, wpre_ref[...], dn, precision=lax.Precision.DEFAULT)

    cb = cb_ref[...]
    cbn = jnp.sum(cb * cb, axis=1)[None, :]
    ii = lax.broadcasted_iota(jnp.int32, (_NK, _NE), 1)
    cb_hi = cb.astype(jnp.bfloat16).astype(jnp.float32)
    cb_mid = (cb - cb_hi).astype(jnp.bfloat16).astype(jnp.float32)
    cb_lo = (cb - cb_hi - cb_mid).astype(jnp.bfloat16).astype(jnp.float32)

    def _level(r, need_residual):
        rn = jnp.sum(r * r, axis=1, keepdims=True)
        sc = lax.dot_general(r, cb, (((1,), (1,)), ((), ())),
                             precision=lax.Precision.DEFAULT)
        dists = rn - 2.0 * sc + cbn
        m = jnp.min(dists, axis=1, keepdims=True)
        idx = jnp.min(jnp.where(dists == m, ii, _NE), axis=1)
        if not need_residual:
            return idx, None
        oh = (ii == idx[:, None]).astype(jnp.float32)
        c = (lax.dot_general(oh, cb_hi, dn, precision=lax.Precision.DEFAULT)
             + lax.dot_general(oh, cb_mid, dn, precision=lax.Precision.DEFAULT)
             + lax.dot_general(oh, cb_lo, dn, precision=lax.Precision.DEFAULT))
        return idx, r - c

    idx0, r1 = _level(e, True)
    idx1, _ = _level(r1, False)

    j = js_ref[pl.program_id(0)]
    A = idx0[None, :]
    Bv = idx1[None, :]
    A0 = A[:, 0:_HALF]
    A1 = A[:, 1:_HALF + 1]
    A2 = A[:, 2:_HALF + 2]
    B0 = Bv[:, 0:_HALF]
    B1 = Bv[:, 1:_HALF + 1]
    Bm1 = jnp.concatenate(
        [jnp.full((1, 1), -3, jnp.int32), Bv[:, 0:_HALF - 1]], axis=1)

    def sel4(v0, v1, v2, v3):
        return jnp.where(j == 0, v0,
               jnp.where(j == 1, v1,
               jnp.where(j == 2, v2, v3)))

    se_ref[...] = (sel4(Bm1, A0, B0, A1) + 3)[None]
    so_ref[...] = (sel4(A0, B0, A1, B1) + 3)[None]
    te_ref[...] = (sel4(A0, B0, A1, B1) + 3)[None]
    to_ref[...] = (sel4(B0, A1, B1, A2) + 3)[None]

    t = lax.broadcasted_iota(jnp.int32, (1, _NK), 1)
    p = t + j
    pi = jnp.where(p == 0, 0, ((p - 1) & 1) + 3)
    shift = jnp.maximum(j - 1, 0) // 2
    po = jnp.where(p == 0, 0, ((p - 1) >> 1) + 3 - shift)
    pi_ref[...] = pi[None]
    po_ref[...] = po[None]


def _vq_pack(js, y4, W_enc, W_pre, codebook):
    i32 = jnp.int32
    out_shapes = (
        jax.ShapeDtypeStruct((_B, 1, _HALF), i32),
        jax.ShapeDtypeStruct((_B, 1, _HALF), i32),
        jax.ShapeDtypeStruct((_B, 1, _HALF), i32),
        jax.ShapeDtypeStruct((_B, 1, _HALF), i32),
        jax.ShapeDtypeStruct((_B, 1, _NK), i32),
        jax.ShapeDtypeStruct((_B, 1, _NK), i32),
    )
    grid = (_B,)
    in_specs = [
        pl.BlockSpec((_B,), lambda k: (0,), memory_space=pltpu.SMEM),
        pl.BlockSpec((_NK, _DQ), lambda k: (k, 0)),
        pl.BlockSpec((_NK, _DQ), lambda k: (k + _B, 0)),
        pl.BlockSpec((_NK, _DQ), lambda k: (k + 2 * _B, 0)),
        pl.BlockSpec((_NK, _DQ), lambda k: (k + 3 * _B, 0)),
        pl.BlockSpec((_D, _D), lambda k: (0, 0)),
        pl.BlockSpec((_D, _VQ), lambda k: (0, 0)),
        pl.BlockSpec((_NE, _VQ), lambda k: (0, 0)),
    ]
    out_specs = (
        pl.BlockSpec((1, 1, _HALF), lambda k: (k, 0, 0)),
        pl.BlockSpec((1, 1, _HALF), lambda k: (k, 0, 0)),
        pl.BlockSpec((1, 1, _HALF), lambda k: (k, 0, 0)),
        pl.BlockSpec((1, 1, _HALF), lambda k: (k, 0, 0)),
        pl.BlockSpec((1, 1, _NK), lambda k: (k, 0, 0)),
        pl.BlockSpec((1, 1, _NK), lambda k: (k, 0, 0)),
    )
    return pl.pallas_call(
        _vq_pack_body,
        grid=grid,
        in_specs=in_specs,
        out_specs=out_specs,
        out_shape=out_shapes,
    )(js, y4, y4, y4, y4, W_enc, W_pre, codebook)


def kernel(x, edge_index, batch, faces, num_vertices, js, W_enc, W_pre, codebook):
    del batch, faces, num_vertices
    src = edge_index[0].astype(jnp.int32)
    dst = edge_index[1].astype(jnp.int32)
    js = js.astype(jnp.int32)

    x4 = jnp.concatenate([x[:, q * _DQ:(q + 1) * _DQ] for q in range(4)],
                         axis=0)
    src4 = jnp.stack([src + q * _N for q in range(4)]).reshape(
        4, _E // _CH, _CH)
    dst2 = dst.reshape(_E // _CH, _CH)
    y4 = _segsum_sc(src4, dst2, x4)

    se, so, te, to, pi, po = _vq_pack(js, y4, W_enc, W_pre, codebook)

    idxt = jax.dtypes.canonicalize_dtype(jnp.int64)
    S = jnp.stack([se.reshape(_B, _HALF), so.reshape(_B, _HALF)],
                  axis=-1).reshape(_B, _NK).astype(idxt)
    T = jnp.stack([te.reshape(_B, _HALF), to.reshape(_B, _HALF)],
                  axis=-1).reshape(_B, _NK).astype(idxt)
    PI = pi.reshape(_B, _NK).astype(idxt)
    PO = po.reshape(_B, _NK).astype(idxt)
    return S, T, PI, PO


# speedup vs baseline: 3.8330x; 1.0029x over previous
import functools

import jax
import jax.numpy as jnp
from jax import lax
from jax.experimental import pallas as pl
from jax.experimental.pallas import tpu as pltpu
from jax.experimental.pallas import tpu_sc as plsc

_N = 16384
_B = 8
_D = 192
_DH = 96
_VQ = 64
_NE = 1024
_NK = _N // _B
_HALF = _NK // 2

_E = 131072
_NSC = 16
_EPT = _E // _NSC
_CH = 128
_NCH = _EPT // _CH
_GRP = 4
_NGRP = _NCH // _GRP
_RPT = _N // _NSC


_DQ = _D // 4


def _segsum_body(src_hbm, dst_hbm, x4_hbm, out_hbm, srcv, dstv, rows, sem,
                 semb, semsc, shared):
    c = lax.axis_index("c")
    s = lax.axis_index("s")
    row0 = s * _RPT

    for t in range(2):
        q = 2 * c + t
        base = q * _N + row0
        for i in range(_RPT // _CH):
            pltpu.sync_copy(x4_hbm.at[pl.ds(base + i * _CH, _CH)],
                            shared.at[pl.ds(row0 + i * _CH, _CH)])
        pltpu.sync_copy(src_hbm.at[q, pl.ds(s * _NCH, _NCH)], srcv)
        if t == 0:
            pltpu.sync_copy(dst_hbm.at[pl.ds(s * _NCH, _NCH)], dstv)
        plsc.subcore_barrier()

        def g_start(g, pbuf, sm):
            for b in range(_GRP):
                pltpu.make_async_copy(x4_hbm.at[srcv.at[g * _GRP + b]],
                                      rows.at[pbuf, b], sm).start()

        def g_wait(g, pbuf, sm):
            for b in range(_GRP):
                pltpu.make_async_copy(x4_hbm.at[srcv.at[g * _GRP + b]],
                                      rows.at[pbuf, b], sm).wait()

        def g_scatter(g, pbuf, sm):
            for b in range(_GRP):
                pltpu.make_async_copy(rows.at[pbuf, b],
                                      shared.at[dstv.at[g * _GRP + b]],
                                      sm).start(add=True)
            for b in range(_GRP):
                pltpu.make_async_copy(rows.at[pbuf, b],
                                      shared.at[dstv.at[g * _GRP + b]],
                                      sm).wait()

        g_start(0, 0, sem)

        def pair(i, carry):
            g0 = 2 * i
            g_start(g0 + 1, 1, semb)
            g_wait(g0, 0, sem)
            g_scatter(g0, 0, semsc)

            @pl.when(i < _NGRP // 2 - 1)
            def _():
                g_start(g0 + 2, 0, sem)

            g_wait(g0 + 1, 1, semb)
            g_scatter(g0 + 1, 1, semsc)
            return carry

        lax.fori_loop(0, _NGRP // 2, pair, 0)
        plsc.subcore_barrier()

        for i in range(_RPT // _CH):
            pltpu.sync_copy(shared.at[pl.ds(row0 + i * _CH, _CH)],
                            out_hbm.at[pl.ds(base + i * _CH, _CH)])
        plsc.subcore_barrier()


@functools.partial(
    pl.kernel,
    out_type=jax.ShapeDtypeStruct((4 * _N, _DQ), jnp.float32),
    mesh=plsc.VectorSubcoreMesh(core_axis_name="c", subcore_axis_name="s"),
    compiler_params=pltpu.CompilerParams(use_tc_tiling_on_sc=False),
    scratch_types=[
        pltpu.VMEM((_NCH, _CH), jnp.int32),
        pltpu.VMEM((_NCH, _CH), jnp.int32),
        pltpu.VMEM((2, _GRP, _CH, _DQ), jnp.float32),
        pltpu.SemaphoreType.DMA,
        pltpu.SemaphoreType.DMA,
        pltpu.SemaphoreType.DMA,
        pltpu.VMEM_SHARED((_N, _DQ), jnp.float32),
    ],
)
def _segsum_sc(src_hbm, dst_hbm, x4_hbm, out_hbm, srcv, dstv, rows, sem, semb,
               semsc, shared):
    _segsum_body(src_hbm, dst_hbm, x4_hbm, out_hbm, srcv, dstv, rows, sem,
                 semb, semsc, shared)


def _vq_pack_body(js_ref, y0_ref, y1_ref, y2_ref, y3_ref, wenc_ref, wpre_ref,
                  cb_ref, se_ref, so_ref, te_ref, to_ref, pi_ref, po_ref):
    dn = (((1,), (0,)), ((), ()))
    acc = None
    for q, yq_ref in enumerate((y0_ref, y1_ref, y2_ref, y3_ref)):
        wq = wenc_ref[q * _DQ:(q + 1) * _DQ, :]
        part = lax.dot_general(yq_ref[...], wq, dn,
                               precision=lax.Precision.DEFAULT)
        acc = part if acc is None else acc + part
    h = jnp.maximum(acc, 0.0)
    e = lax.dot_general(h, wpre_ref[...], dn, precision=lax.Precision.DEFAULT)

    cb = cb_ref[...]
    cbn = jnp.sum(cb * cb, axis=1)[None, :]
    ii = lax.broadcasted_iota(jnp.int32, (_NK, _NE), 1)
    cb_hi = cb.astype(jnp.bfloat16).astype(jnp.float32)
    cb_mid = (cb - cb_hi).astype(jnp.bfloat16).astype(jnp.float32)
    cb_lo = (cb - cb_hi - cb_mid).astype(jnp.bfloat16).astype(jnp.float32)

    def _level(r, need_residual):
        rn = jnp.sum(r * r, axis=1, keepdims=True)
        sc = lax.dot_general(r, cb, (((1,), (1,)), ((), ())),
                             precision=lax.Precision.DEFAULT)
        dists = rn - 2.0 * sc + cbn
        m = jnp.min(dists, axis=1, keepdims=True)
        idx = jnp.min(jnp.where(dists == m, ii, _NE), axis=1)
        if not need_residual:
            return idx, None
        oh = (ii == idx[:, None]).astype(jnp.float32)
        c = (lax.dot_general(oh, cb_hi, dn, precision=lax.Precision.DEFAULT)
             + lax.dot_general(oh, cb_mid, dn, precision=lax.Precision.DEFAULT)
             + lax.dot_general(oh, cb_lo, dn, precision=lax.Precision.DEFAULT))
        return idx, r - c

    idx0, r1 = _level(e, True)
    idx1, _ = _level(r1, False)

    j = js_ref[pl.program_id(0)]
    A = idx0[None, :]
    Bv = idx1[None, :]
    A0 = A[:, 0:_HALF]
    A1 = A[:, 1:_HALF + 1]
    A2 = A[:, 2:_HALF + 2]
    B0 = Bv[:, 0:_HALF]
    B1 = Bv[:, 1:_HALF + 1]
    Bm1 = jnp.concatenate(
        [jnp.full((1, 1), -3, jnp.int32), Bv[:, 0:_HALF - 1]], axis=1)

    def sel4(v0, v1, v2, v3):
        return jnp.where(j == 0, v0,
               jnp.where(j == 1, v1,
               jnp.where(j == 2, v2, v3)))

    se_ref[...] = (sel4(Bm1, A0, B0, A1) + 3)[None]
    so_ref[...] = (sel4(A0, B0, A1, B1) + 3)[None]
    te_ref[...] = (sel4(A0, B0, A1, B1) + 3)[None]
    to_ref[...] = (sel4(B0, A1, B1, A2) + 3)[None]

    t = lax.broadcasted_iota(jnp.int32, (1, _NK), 1)
    p = t + j
    pi = jnp.where(p == 0, 0, ((p - 1) & 1) + 3)
    shift = jnp.maximum(j - 1, 0) // 2
    po = jnp.where(p == 0, 0, ((p - 1) >> 1) + 3 - shift)
    pi_ref[...] = pi[None]
    po_ref[...] = po[None]


def _vq_pack(js, y4, W_enc, W_pre, codebook):
    i32 = jnp.int32
    out_shapes = (
        jax.ShapeDtypeStruct((_B, 1, _HALF), i32),
        jax.ShapeDtypeStruct((_B, 1, _HALF), i32),
        jax.ShapeDtypeStruct((_B, 1, _HALF), i32),
        jax.ShapeDtypeStruct((_B, 1, _HALF), i32),
        jax.ShapeDtypeStruct((_B, 1, _NK), i32),
        jax.ShapeDtypeStruct((_B, 1, _NK), i32),
    )
    grid = (_B,)
    in_specs = [
        pl.BlockSpec((_B,), lambda k: (0,), memory_space=pltpu.SMEM),
        pl.BlockSpec((_NK, _DQ), lambda k: (k, 0)),
        pl.BlockSpec((_NK, _DQ), lambda k: (k + _B, 0)),
        pl.BlockSpec((_NK, _DQ), lambda k: (k + 2 * _B, 0)),
        pl.BlockSpec((_NK, _DQ), lambda k: (k + 3 * _B, 0)),
        pl.BlockSpec((_D, _D), lambda k: (0, 0)),
        pl.BlockSpec((_D, _VQ), lambda k: (0, 0)),
        pl.BlockSpec((_NE, _VQ), lambda k: (0, 0)),
    ]
    out_specs = (
        pl.BlockSpec((1, 1, _HALF), lambda k: (k, 0, 0)),
        pl.BlockSpec((1, 1, _HALF), lambda k: (k, 0, 0)),
        pl.BlockSpec((1, 1, _HALF), lambda k: (k, 0, 0)),
        pl.BlockSpec((1, 1, _HALF), lambda k: (k, 0, 0)),
        pl.BlockSpec((1, 1, _NK), lambda k: (k, 0, 0)),
        pl.BlockSpec((1, 1, _NK), lambda k: (k, 0, 0)),
    )
    return pl.pallas_call(
        _vq_pack_body,
        grid=grid,
        in_specs=in_specs,
        out_specs=out_specs,
        out_shape=out_shapes,
    )(js, y4, y4, y4, y4, W_enc, W_pre, codebook)


def kernel(x, edge_index, batch, faces, num_vertices, js, W_enc, W_pre, codebook):
    del batch, faces, num_vertices
    src = edge_index[0].astype(jnp.int32)
    dst = edge_index[1].astype(jnp.int32)
    js = js.astype(jnp.int32)

    x4 = jnp.concatenate([x[:, q * _DQ:(q + 1) * _DQ] for q in range(4)],
                         axis=0)
    src4 = jnp.stack([src + q * _N for q in range(4)]).reshape(
        4, _E // _CH, _CH)
    dst2 = dst.reshape(_E // _CH, _CH)
    y4 = _segsum_sc(src4, dst2, x4)

    se, so, te, to, pi, po = _vq_pack(js, y4, W_enc, W_pre, codebook)

    idxt = jax.dtypes.canonicalize_dtype(jnp.int64)
    S = jnp.stack([se.reshape(_B, _HALF), so.reshape(_B, _HALF)],
                  axis=-1).reshape(_B, _NK).astype(idxt)
    T = jnp.stack([te.reshape(_B, _HALF), to.reshape(_B, _HALF)],
                  axis=-1).reshape(_B, _NK).astype(idxt)
    PI = pi.reshape(_B, _NK).astype(idxt)
    PO = po.reshape(_B, _NK).astype(idxt)
    return S, T, PI, PO


# speedup vs baseline: 3.9750x; 1.0371x over previous
import functools

import jax
import jax.numpy as jnp
from jax import lax
from jax.experimental import pallas as pl
from jax.experimental.pallas import tpu as pltpu
from jax.experimental.pallas import tpu_sc as plsc

_N = 16384
_B = 8
_D = 192
_DH = 96
_VQ = 64
_NE = 1024
_NK = _N // _B
_HALF = _NK // 2

_E = 131072
_NSC = 16
_EPT = _E // _NSC
_CH = 128
_NCH = _EPT // _CH
_GRP = 4
_NGRP = _NCH // _GRP
_RPT = _N // _NSC


_DQ = _D // 4


def _segsum_body(src_hbm, dst_hbm, x4_hbm, out_hbm, srcv, dstv, rows, sem,
                 semb, semsc, shared):
    c = lax.axis_index("c")
    s = lax.axis_index("s")
    row0 = s * _RPT

    for t in range(2):
        q = 2 * c + t
        base = q * _N + row0
        for i in range(_RPT // _CH):
            pltpu.sync_copy(x4_hbm.at[pl.ds(base + i * _CH, _CH)],
                            shared.at[pl.ds(row0 + i * _CH, _CH)])
        pltpu.sync_copy(src_hbm.at[q, pl.ds(s * _NCH, _NCH)], srcv)
        if t == 0:
            pltpu.sync_copy(dst_hbm.at[pl.ds(s * _NCH, _NCH)], dstv)
        plsc.subcore_barrier()

        def g_start(g, pbuf, sm):
            for b in range(_GRP):
                pltpu.make_async_copy(x4_hbm.at[srcv.at[g * _GRP + b]],
                                      rows.at[pbuf, b], sm).start()

        def g_wait(g, pbuf, sm):
            for b in range(_GRP):
                pltpu.make_async_copy(x4_hbm.at[srcv.at[g * _GRP + b]],
                                      rows.at[pbuf, b], sm).wait()

        def g_scatter(g, pbuf, sm):
            for b in range(_GRP):
                pltpu.make_async_copy(rows.at[pbuf, b],
                                      shared.at[dstv.at[g * _GRP + b]],
                                      sm).start(add=True)
            for b in range(_GRP):
                pltpu.make_async_copy(rows.at[pbuf, b],
                                      shared.at[dstv.at[g * _GRP + b]],
                                      sm).wait()

        g_start(0, 0, sem)

        def pair(i, carry):
            g0 = 2 * i
            g_start(g0 + 1, 1, semb)
            g_wait(g0, 0, sem)
            g_scatter(g0, 0, semsc)

            @pl.when(i < _NGRP // 2 - 1)
            def _():
                g_start(g0 + 2, 0, sem)

            g_wait(g0 + 1, 1, semb)
            g_scatter(g0 + 1, 1, semsc)
            return carry

        lax.fori_loop(0, _NGRP // 2, pair, 0)
        plsc.subcore_barrier()

        for i in range(_RPT // _CH):
            pltpu.sync_copy(shared.at[pl.ds(row0 + i * _CH, _CH)],
                            out_hbm.at[pl.ds(base + i * _CH, _CH)])
        plsc.subcore_barrier()


@functools.partial(
    pl.kernel,
    out_type=jax.ShapeDtypeStruct((4 * _N, _DQ), jnp.float32),
    mesh=plsc.VectorSubcoreMesh(core_axis_name="c", subcore_axis_name="s"),
    compiler_params=pltpu.CompilerParams(use_tc_tiling_on_sc=False),
    scratch_types=[
        pltpu.VMEM((_NCH, _CH), jnp.int32),
        pltpu.VMEM((_NCH, _CH), jnp.int32),
        pltpu.VMEM((2, _GRP, _CH, _DQ), jnp.float32),
        pltpu.SemaphoreType.DMA,
        pltpu.SemaphoreType.DMA,
        pltpu.SemaphoreType.DMA,
        pltpu.VMEM_SHARED((_N, _DQ), jnp.float32),
    ],
)
def _segsum_sc(src_hbm, dst_hbm, x4_hbm, out_hbm, srcv, dstv, rows, sem, semb,
               semsc, shared):
    _segsum_body(src_hbm, dst_hbm, x4_hbm, out_hbm, srcv, dstv, rows, sem,
                 semb, semsc, shared)


def _vq_pack_body(js_ref, y4_ref, wenc_ref, wpre_ref, cb_ref,
                  se_ref, so_ref, te_ref, to_ref, pi_ref, po_ref,
                  y0v, y1v, y2v, y3v, ysem):
    k = pl.program_id(0)
    yvs = (y0v, y1v, y2v, y3v)
    copies = [pltpu.async_copy(y4_ref.at[pl.ds(q * _N + k * _NK, _NK)], yv,
                               ysem) for q, yv in enumerate(yvs)]
    for cpy in copies:
        cpy.wait()
    dn = (((1,), (0,)), ((), ()))
    acc = None
    for q, yv in enumerate(yvs):
        wq = wenc_ref[q * _DQ:(q + 1) * _DQ, :]
        part = lax.dot_general(yv[...], wq, dn,
                               precision=lax.Precision.DEFAULT)
        acc = part if acc is None else acc + part
    h = jnp.maximum(acc, 0.0)
    e = lax.dot_general(h, wpre_ref[...], dn, precision=lax.Precision.DEFAULT)

    cb = cb_ref[...]
    cbn = jnp.sum(cb * cb, axis=1)[None, :]
    ii = lax.broadcasted_iota(jnp.int32, (_NK, _NE), 1)
    bf = jnp.bfloat16
    cb_hi = cb.astype(bf)
    cb_mid = (cb - cb_hi.astype(jnp.float32)).astype(bf)
    cb_lo = (cb - cb_hi.astype(jnp.float32)
             - cb_mid.astype(jnp.float32)).astype(bf)
    cb3 = jnp.concatenate([cb_hi, cb_mid, cb_lo], axis=1)

    def _level(r, need_residual):
        rn = jnp.sum(r * r, axis=1, keepdims=True)
        sc = lax.dot_general(r, cb, (((1,), (1,)), ((), ())),
                             precision=lax.Precision.DEFAULT)
        dists = rn - 2.0 * sc + cbn
        m = jnp.min(dists, axis=1, keepdims=True)
        idx = jnp.min(jnp.where(dists == m, ii, _NE), axis=1)
        if not need_residual:
            return idx, None
        oh = (ii == idx[:, None]).astype(bf)
        c3 = lax.dot_general(oh, cb3, dn, precision=lax.Precision.DEFAULT,
                             preferred_element_type=jnp.float32)
        c = ((c3[:, 0:_VQ] + c3[:, _VQ:2 * _VQ]) + c3[:, 2 * _VQ:3 * _VQ])
        return idx, r - c

    idx0, r1 = _level(e, True)
    idx1, _ = _level(r1, False)

    j = js_ref[pl.program_id(0)]
    A = idx0[None, :]
    Bv = idx1[None, :]
    A0 = A[:, 0:_HALF]
    A1 = A[:, 1:_HALF + 1]
    A2 = A[:, 2:_HALF + 2]
    B0 = Bv[:, 0:_HALF]
    B1 = Bv[:, 1:_HALF + 1]
    Bm1 = jnp.concatenate(
        [jnp.full((1, 1), -3, jnp.int32), Bv[:, 0:_HALF - 1]], axis=1)

    def sel4(v0, v1, v2, v3):
        return jnp.where(j == 0, v0,
               jnp.where(j == 1, v1,
               jnp.where(j == 2, v2, v3)))

    se_ref[...] = (sel4(Bm1, A0, B0, A1) + 3)[None]
    so_ref[...] = (sel4(A0, B0, A1, B1) + 3)[None]
    te_ref[...] = (sel4(A0, B0, A1, B1) + 3)[None]
    to_ref[...] = (sel4(B0, A1, B1, A2) + 3)[None]

    t = lax.broadcasted_iota(jnp.int32, (1, _NK), 1)
    p = t + j
    pi = jnp.where(p == 0, 0, ((p - 1) & 1) + 3)
    shift = jnp.maximum(j - 1, 0) // 2
    po = jnp.where(p == 0, 0, ((p - 1) >> 1) + 3 - shift)
    pi_ref[...] = pi[None]
    po_ref[...] = po[None]


def _vq_pack(js, y4, W_enc, W_pre, codebook):
    i32 = jnp.int32
    out_shapes = (
        jax.ShapeDtypeStruct((_B, 1, _HALF), i32),
        jax.ShapeDtypeStruct((_B, 1, _HALF), i32),
        jax.ShapeDtypeStruct((_B, 1, _HALF), i32),
        jax.ShapeDtypeStruct((_B, 1, _HALF), i32),
        jax.ShapeDtypeStruct((_B, 1, _NK), i32),
        jax.ShapeDtypeStruct((_B, 1, _NK), i32),
    )
    grid = (_B,)
    in_specs = [
        pl.BlockSpec((_B,), lambda k: (0,), memory_space=pltpu.SMEM),
        pl.BlockSpec(memory_space=pl.ANY),
        pl.BlockSpec((_D, _D), lambda k: (0, 0)),
        pl.BlockSpec((_D, _VQ), lambda k: (0, 0)),
        pl.BlockSpec((_NE, _VQ), lambda k: (0, 0)),
    ]
    out_specs = (
        pl.BlockSpec((1, 1, _HALF), lambda k: (k, 0, 0)),
        pl.BlockSpec((1, 1, _HALF), lambda k: (k, 0, 0)),
        pl.BlockSpec((1, 1, _HALF), lambda k: (k, 0, 0)),
        pl.BlockSpec((1, 1, _HALF), lambda k: (k, 0, 0)),
        pl.BlockSpec((1, 1, _NK), lambda k: (k, 0, 0)),
        pl.BlockSpec((1, 1, _NK), lambda k: (k, 0, 0)),
    )
    return pl.pallas_call(
        _vq_pack_body,
        grid=grid,
        in_specs=in_specs,
        out_specs=out_specs,
        out_shape=out_shapes,
        scratch_shapes=[pltpu.VMEM((_NK, _DQ), jnp.float32)] * 4
                       + [pltpu.SemaphoreType.DMA],
    )(js, y4, W_enc, W_pre, codebook)


def kernel(x, edge_index, batch, faces, num_vertices, js, W_enc, W_pre, codebook):
    del batch, faces, num_vertices
    src = edge_index[0].astype(jnp.int32)
    dst = edge_index[1].astype(jnp.int32)
    js = js.astype(jnp.int32)

    x4 = jnp.concatenate([x[:, q * _DQ:(q + 1) * _DQ] for q in range(4)],
                         axis=0)
    src4 = jnp.stack([src + q * _N for q in range(4)]).reshape(
        4, _E // _CH, _CH)
    dst2 = dst.reshape(_E // _CH, _CH)
    y4 = _segsum_sc(src4, dst2, x4)

    se, so, te, to, pi, po = _vq_pack(js, y4, W_enc, W_pre, codebook)

    idxt = jax.dtypes.canonicalize_dtype(jnp.int64)
    S = jnp.stack([se.reshape(_B, _HALF), so.reshape(_B, _HALF)],
                  axis=-1).reshape(_B, _NK).astype(idxt)
    T = jnp.stack([te.reshape(_B, _HALF), to.reshape(_B, _HALF)],
                  axis=-1).reshape(_B, _NK).astype(idxt)
    PI = pi.reshape(_B, _NK).astype(idxt)
    PO = po.reshape(_B, _NK).astype(idxt)
    return S, T, PI, PO


# speedup vs baseline: 4.1863x; 1.0531x over previous
import functools

import jax
import jax.numpy as jnp
from jax import lax
from jax.experimental import pallas as pl
from jax.experimental.pallas import tpu as pltpu
from jax.experimental.pallas import tpu_sc as plsc

_N = 16384
_B = 8
_D = 192
_DH = 96
_VQ = 64
_NE = 1024
_NK = _N // _B
_HALF = _NK // 2

_E = 131072
_NSC = 16
_EPT = _E // _NSC
_CH = 128
_NCH = _EPT // _CH
_GRP = 4
_NGRP = _NCH // _GRP
_RPT = _N // _NSC


_DQ = _D // 4


def _segsum_body(src_hbm, dst_hbm, x4_hbm, out_hbm, srcv, dstv, rows, sem,
                 semb, semsc, shared):
    c = lax.axis_index("c")
    s = lax.axis_index("s")
    row0 = s * _RPT

    for t in range(2):
        q = 2 * c + t
        base = q * _N + row0
        for i in range(_RPT // _CH):
            pltpu.make_async_copy(x4_hbm.at[pl.ds(base + i * _CH, _CH)],
                                  shared.at[pl.ds(row0 + i * _CH, _CH)],
                                  semb).start()
        pltpu.sync_copy(src_hbm.at[q, pl.ds(s * _NCH, _NCH)], srcv)
        if t == 0:
            pltpu.sync_copy(dst_hbm.at[pl.ds(s * _NCH, _NCH)], dstv)

        def g_start(g, pbuf, sm):
            for b in range(_GRP):
                pltpu.make_async_copy(x4_hbm.at[srcv.at[g * _GRP + b]],
                                      rows.at[pbuf, b], sm).start()

        def g_wait(g, pbuf, sm):
            for b in range(_GRP):
                pltpu.make_async_copy(x4_hbm.at[srcv.at[g * _GRP + b]],
                                      rows.at[pbuf, b], sm).wait()

        def g_scatter(g, pbuf, sm):
            for b in range(_GRP):
                pltpu.make_async_copy(rows.at[pbuf, b],
                                      shared.at[dstv.at[g * _GRP + b]],
                                      sm).start(add=True)
            for b in range(_GRP):
                pltpu.make_async_copy(rows.at[pbuf, b],
                                      shared.at[dstv.at[g * _GRP + b]],
                                      sm).wait()

        g_start(0, 0, sem)
        for i in range(_RPT // _CH):
            pltpu.make_async_copy(x4_hbm.at[pl.ds(base + i * _CH, _CH)],
                                  shared.at[pl.ds(row0 + i * _CH, _CH)],
                                  semb).wait()
        plsc.subcore_barrier()

        def pair(i, carry):
            g0 = 2 * i
            g_start(g0 + 1, 1, semb)
            g_wait(g0, 0, sem)
            g_scatter(g0, 0, semsc)

            @pl.when(i < _NGRP // 2 - 1)
            def _():
                g_start(g0 + 2, 0, sem)

            g_wait(g0 + 1, 1, semb)
            g_scatter(g0 + 1, 1, semsc)
            return carry

        lax.fori_loop(0, _NGRP // 2, pair, 0)
        plsc.subcore_barrier()

        for i in range(_RPT // _CH):
            pltpu.make_async_copy(shared.at[pl.ds(row0 + i * _CH, _CH)],
                                  out_hbm.at[pl.ds(base + i * _CH, _CH)],
                                  semb).start()
        for i in range(_RPT // _CH):
            pltpu.make_async_copy(shared.at[pl.ds(row0 + i * _CH, _CH)],
                                  out_hbm.at[pl.ds(base + i * _CH, _CH)],
                                  semb).wait()


@functools.partial(
    pl.kernel,
    out_type=jax.ShapeDtypeStruct((4 * _N, _DQ), jnp.float32),
    mesh=plsc.VectorSubcoreMesh(core_axis_name="c", subcore_axis_name="s"),
    compiler_params=pltpu.CompilerParams(use_tc_tiling_on_sc=False),
    scratch_types=[
        pltpu.VMEM((_NCH, _CH), jnp.int32),
        pltpu.VMEM((_NCH, _CH), jnp.int32),
        pltpu.VMEM((2, _GRP, _CH, _DQ), jnp.float32),
        pltpu.SemaphoreType.DMA,
        pltpu.SemaphoreType.DMA,
        pltpu.SemaphoreType.DMA,
        pltpu.VMEM_SHARED((_N, _DQ), jnp.float32),
    ],
)
def _segsum_sc(src_hbm, dst_hbm, x4_hbm, out_hbm, srcv, dstv, rows, sem, semb,
               semsc, shared):
    _segsum_body(src_hbm, dst_hbm, x4_hbm, out_hbm, srcv, dstv, rows, sem,
                 semb, semsc, shared)


def _vq_pack_body(js_ref, y4_ref, wenc_ref, wpre_ref, cb_ref,
                  se_ref, so_ref, te_ref, to_ref, pi_ref, po_ref,
                  y0v, y1v, y2v, y3v, ysem):
    k = pl.program_id(0)
    yvs = (y0v, y1v, y2v, y3v)
    copies = [pltpu.async_copy(y4_ref.at[pl.ds(q * _N + k * _NK, _NK)], yv,
                               ysem) for q, yv in enumerate(yvs)]
    for cpy in copies:
        cpy.wait()
    dn = (((1,), (0,)), ((), ()))
    acc = None
    for q, yv in enumerate(yvs):
        wq = wenc_ref[q * _DQ:(q + 1) * _DQ, :]
        part = lax.dot_general(yv[...], wq, dn,
                               precision=lax.Precision.DEFAULT)
        acc = part if acc is None else acc + part
    h = jnp.maximum(acc, 0.0)
    e = lax.dot_general(h, wpre_ref[...], dn, precision=lax.Precision.DEFAULT)

    cb = cb_ref[...]
    cbn = jnp.sum(cb * cb, axis=1)[None, :]
    ii = lax.broadcasted_iota(jnp.int32, (_NK, _NE), 1)
    bf = jnp.bfloat16
    cb_hi = cb.astype(bf)
    cb_mid = (cb - cb_hi.astype(jnp.float32)).astype(bf)
    cb_lo = (cb - cb_hi.astype(jnp.float32)
             - cb_mid.astype(jnp.float32)).astype(bf)
    cb3 = jnp.concatenate([cb_hi, cb_mid, cb_lo], axis=1)

    def _level(r, need_residual):
        rn = jnp.sum(r * r, axis=1, keepdims=True)
        sc = lax.dot_general(r, cb, (((1,), (1,)), ((), ())),
                             precision=lax.Precision.DEFAULT)
        dists = rn - 2.0 * sc + cbn
        m = jnp.min(dists, axis=1, keepdims=True)
        idx = jnp.min(jnp.where(dists == m, ii, _NE), axis=1)
        if not need_residual:
            return idx, None
        oh = (ii == idx[:, None]).astype(bf)
        c3 = lax.dot_general(oh, cb3, dn, precision=lax.Precision.DEFAULT,
                             preferred_element_type=jnp.float32)
        c = ((c3[:, 0:_VQ] + c3[:, _VQ:2 * _VQ]) + c3[:, 2 * _VQ:3 * _VQ])
        return idx, r - c

    idx0, r1 = _level(e, True)
    idx1, _ = _level(r1, False)

    j = js_ref[pl.program_id(0)]
    A = idx0[None, :]
    Bv = idx1[None, :]
    A0 = A[:, 0:_HALF]
    A1 = A[:, 1:_HALF + 1]
    A2 = A[:, 2:_HALF + 2]
    B0 = Bv[:, 0:_HALF]
    B1 = Bv[:, 1:_HALF + 1]
    Bm1 = jnp.concatenate(
        [jnp.full((1, 1), -3, jnp.int32), Bv[:, 0:_HALF - 1]], axis=1)

    def sel4(v0, v1, v2, v3):
        return jnp.where(j == 0, v0,
               jnp.where(j == 1, v1,
               jnp.where(j == 2, v2, v3)))

    se_ref[...] = (sel4(Bm1, A0, B0, A1) + 3)[None]
    so_ref[...] = (sel4(A0, B0, A1, B1) + 3)[None]
    te_ref[...] = (sel4(A0, B0, A1, B1) + 3)[None]
    to_ref[...] = (sel4(B0, A1, B1, A2) + 3)[None]

    t = lax.broadcasted_iota(jnp.int32, (1, _NK), 1)
    p = t + j
    pi = jnp.where(p == 0, 0, ((p - 1) & 1) + 3)
    shift = jnp.maximum(j - 1, 0) // 2
    po = jnp.where(p == 0, 0, ((p - 1) >> 1) + 3 - shift)
    pi_ref[...] = pi[None]
    po_ref[...] = po[None]


def _vq_pack(js, y4, W_enc, W_pre, codebook):
    i32 = jnp.int32
    out_shapes = (
        jax.ShapeDtypeStruct((_B, 1, _HALF), i32),
        jax.ShapeDtypeStruct((_B, 1, _HALF), i32),
        jax.ShapeDtypeStruct((_B, 1, _HALF), i32),
        jax.ShapeDtypeStruct((_B, 1, _HALF), i32),
        jax.ShapeDtypeStruct((_B, 1, _NK), i32),
        jax.ShapeDtypeStruct((_B, 1, _NK), i32),
    )
    grid = (_B,)
    in_specs = [
        pl.BlockSpec((_B,), lambda k: (0,), memory_space=pltpu.SMEM),
        pl.BlockSpec(memory_space=pl.ANY),
        pl.BlockSpec((_D, _D), lambda k: (0, 0)),
        pl.BlockSpec((_D, _VQ), lambda k: (0, 0)),
        pl.BlockSpec((_NE, _VQ), lambda k: (0, 0)),
    ]
    out_specs = (
        pl.BlockSpec((1, 1, _HALF), lambda k: (k, 0, 0)),
        pl.BlockSpec((1, 1, _HALF), lambda k: (k, 0, 0)),
        pl.BlockSpec((1, 1, _HALF), lambda k: (k, 0, 0)),
        pl.BlockSpec((1, 1, _HALF), lambda k: (k, 0, 0)),
        pl.BlockSpec((1, 1, _NK), lambda k: (k, 0, 0)),
        pl.BlockSpec((1, 1, _NK), lambda k: (k, 0, 0)),
    )
    return pl.pallas_call(
        _vq_pack_body,
        grid=grid,
        in_specs=in_specs,
        out_specs=out_specs,
        out_shape=out_shapes,
        scratch_shapes=[pltpu.VMEM((_NK, _DQ), jnp.float32)] * 4
                       + [pltpu.SemaphoreType.DMA],
    )(js, y4, W_enc, W_pre, codebook)


def kernel(x, edge_index, batch, faces, num_vertices, js, W_enc, W_pre, codebook):
    del batch, faces, num_vertices
    src = edge_index[0].astype(jnp.int32)
    dst = edge_index[1].astype(jnp.int32)
    js = js.astype(jnp.int32)

    x4 = jnp.concatenate([x[:, q * _DQ:(q + 1) * _DQ] for q in range(4)],
                         axis=0)
    src4 = jnp.stack([src + q * _N for q in range(4)]).reshape(
        4, _E // _CH, _CH)
    dst2 = dst.reshape(_E // _CH, _CH)
    y4 = _segsum_sc(src4, dst2, x4)

    se, so, te, to, pi, po = _vq_pack(js, y4, W_enc, W_pre, codebook)

    idxt = jax.dtypes.canonicalize_dtype(jnp.int64)
    S = jnp.stack([se.reshape(_B, _HALF), so.reshape(_B, _HALF)],
                  axis=-1).reshape(_B, _NK).astype(idxt)
    T = jnp.stack([te.reshape(_B, _HALF), to.reshape(_B, _HALF)],
                  axis=-1).reshape(_B, _NK).astype(idxt)
    PI = pi.reshape(_B, _NK).astype(idxt)
    PO = po.reshape(_B, _NK).astype(idxt)
    return S, T, PI, PO


# speedup vs baseline: 4.4170x; 1.0551x over previous
import functools

import jax
import jax.numpy as jnp
from jax import lax
from jax.experimental import pallas as pl
from jax.experimental.pallas import tpu as pltpu
from jax.experimental.pallas import tpu_sc as plsc

_N = 16384
_B = 8
_D = 192
_DH = 96
_VQ = 64
_NE = 1024
_NK = _N // _B
_HALF = _NK // 2

_E = 131072
_NSC = 16
_EPT = _E // _NSC
_CH = 128
_NCH = _EPT // _CH
_GRP = 4
_NGRP = _NCH // _GRP
_RPT = _N // _NSC


_DQ = _D // 4


def _segsum_body(src_hbm, dst_hbm, x4_hbm, out_hbm, srcv, dstv, rows, sem,
                 semb, semsc, shared):
    c = lax.axis_index("c")
    s = lax.axis_index("s")
    row0 = s * _RPT

    for t in range(2):
        q = 2 * c + t
        base = q * _N + row0
        for i in range(_RPT // _CH):
            pltpu.make_async_copy(x4_hbm.at[pl.ds(base + i * _CH, _CH)],
                                  shared.at[pl.ds(row0 + i * _CH, _CH)],
                                  semb).start()
        pltpu.sync_copy(src_hbm.at[q, pl.ds(s * _NCH, _NCH)], srcv)
        if t == 0:
            pltpu.sync_copy(dst_hbm.at[pl.ds(s * _NCH, _NCH)], dstv)

        def g_start(g, pbuf, sm):
            for b in range(_GRP):
                pltpu.make_async_copy(x4_hbm.at[srcv.at[g * _GRP + b]],
                                      rows.at[pbuf, b], sm).start()

        def g_wait(g, pbuf, sm):
            for b in range(_GRP):
                pltpu.make_async_copy(x4_hbm.at[srcv.at[g * _GRP + b]],
                                      rows.at[pbuf, b], sm).wait()

        def g_scatter(g, pbuf, sm):
            for b in range(_GRP):
                pltpu.make_async_copy(rows.at[pbuf, b],
                                      shared.at[dstv.at[g * _GRP + b]],
                                      sm).start(add=True)
            for b in range(_GRP):
                pltpu.make_async_copy(rows.at[pbuf, b],
                                      shared.at[dstv.at[g * _GRP + b]],
                                      sm).wait()

        g_start(0, 0, sem)
        for i in range(_RPT // _CH):
            pltpu.make_async_copy(x4_hbm.at[pl.ds(base + i * _CH, _CH)],
                                  shared.at[pl.ds(row0 + i * _CH, _CH)],
                                  semb).wait()
        plsc.subcore_barrier()

        def pair(i, carry):
            g0 = 2 * i
            g_start(g0 + 1, 1, semb)
            g_wait(g0, 0, sem)
            g_scatter(g0, 0, semsc)

            @pl.when(i < _NGRP // 2 - 1)
            def _():
                g_start(g0 + 2, 0, sem)

            g_wait(g0 + 1, 1, semb)
            g_scatter(g0 + 1, 1, semsc)
            return carry

        lax.fori_loop(0, _NGRP // 2, pair, 0)
        plsc.subcore_barrier()

        for i in range(_RPT // _CH):
            pltpu.make_async_copy(shared.at[pl.ds(row0 + i * _CH, _CH)],
                                  out_hbm.at[pl.ds(base + i * _CH, _CH)],
                                  semb).start()
        for i in range(_RPT // _CH):
            pltpu.make_async_copy(shared.at[pl.ds(row0 + i * _CH, _CH)],
                                  out_hbm.at[pl.ds(base + i * _CH, _CH)],
                                  semb).wait()


@functools.partial(
    pl.kernel,
    out_type=jax.ShapeDtypeStruct((4 * _N, _DQ), jnp.float32),
    mesh=plsc.VectorSubcoreMesh(core_axis_name="c", subcore_axis_name="s"),
    compiler_params=pltpu.CompilerParams(use_tc_tiling_on_sc=False),
    scratch_types=[
        pltpu.VMEM((_NCH, _CH), jnp.int32),
        pltpu.VMEM((_NCH, _CH), jnp.int32),
        pltpu.VMEM((2, _GRP, _CH, _DQ), jnp.float32),
        pltpu.SemaphoreType.DMA,
        pltpu.SemaphoreType.DMA,
        pltpu.SemaphoreType.DMA,
        pltpu.VMEM_SHARED((_N, _DQ), jnp.float32),
    ],
)
def _segsum_sc(src_hbm, dst_hbm, x4_hbm, out_hbm, srcv, dstv, rows, sem, semb,
               semsc, shared):
    _segsum_body(src_hbm, dst_hbm, x4_hbm, out_hbm, srcv, dstv, rows, sem,
                 semb, semsc, shared)


def _vq_pack_body(js_ref, y4_ref, wenc_ref, wpre_ref, cb_ref,
                  se_ref, so_ref, te_ref, to_ref, pi_ref, po_ref,
                  ybuf, ysem):
    k = pl.program_id(0)
    par = k % 2

    def start4(kk, pbuf):
        for q in range(4):
            pltpu.make_async_copy(
                y4_ref.at[pl.ds(q * _N + kk * _NK, _NK)],
                ybuf.at[pbuf, q], ysem).start()

    @pl.when(k == 0)
    def _():
        start4(k, 0)

    for q in range(4):
        pltpu.make_async_copy(y4_ref.at[pl.ds(0, _NK)], ybuf.at[0, q],
                              ysem).wait()

    @pl.when((k < _B - 1) & (par == 0))
    def _():
        start4(k + 1, 1)

    @pl.when((k < _B - 1) & (par == 1))
    def _():
        start4(k + 1, 0)

    dn = (((1,), (0,)), ((), ()))
    acc = None
    for q in range(4):
        yv = jnp.where(par == 0, ybuf[0, q], ybuf[1, q])
        wq = wenc_ref[q * _DQ:(q + 1) * _DQ, :]
        part = lax.dot_general(yv, wq, dn,
                               precision=lax.Precision.DEFAULT)
        acc = part if acc is None else acc + part
    h = jnp.maximum(acc, 0.0)
    e = lax.dot_general(h, wpre_ref[...], dn, precision=lax.Precision.DEFAULT)

    cb = cb_ref[...]
    cbn = jnp.sum(cb * cb, axis=1)[None, :]
    ii = lax.broadcasted_iota(jnp.int32, (_NK, _NE), 1)
    bf = jnp.bfloat16
    cb_hi = cb.astype(bf)
    cb_mid = (cb - cb_hi.astype(jnp.float32)).astype(bf)
    cb_lo = (cb - cb_hi.astype(jnp.float32)
             - cb_mid.astype(jnp.float32)).astype(bf)
    cb3 = jnp.concatenate([cb_hi, cb_mid, cb_lo], axis=1)

    def _level(r, need_residual):
        rn = jnp.sum(r * r, axis=1, keepdims=True)
        sc = lax.dot_general(r, cb, (((1,), (1,)), ((), ())),
                             precision=lax.Precision.DEFAULT)
        dists = rn - 2.0 * sc + cbn
        m = jnp.min(dists, axis=1, keepdims=True)
        idx = jnp.min(jnp.where(dists == m, ii, _NE), axis=1)
        if not need_residual:
            return idx, None
        oh = (ii == idx[:, None]).astype(bf)
        c3 = lax.dot_general(oh, cb3, dn, precision=lax.Precision.DEFAULT,
                             preferred_element_type=jnp.float32)
        c = ((c3[:, 0:_VQ] + c3[:, _VQ:2 * _VQ]) + c3[:, 2 * _VQ:3 * _VQ])
        return idx, r - c

    idx0, r1 = _level(e, True)
    idx1, _ = _level(r1, False)

    j = js_ref[pl.program_id(0)]
    A = idx0[None, :]
    Bv = idx1[None, :]
    A0 = A[:, 0:_HALF]
    A1 = A[:, 1:_HALF + 1]
    A2 = A[:, 2:_HALF + 2]
    B0 = Bv[:, 0:_HALF]
    B1 = Bv[:, 1:_HALF + 1]
    Bm1 = jnp.concatenate(
        [jnp.full((1, 1), -3, jnp.int32), Bv[:, 0:_HALF - 1]], axis=1)

    def sel4(v0, v1, v2, v3):
        return jnp.where(j == 0, v0,
               jnp.where(j == 1, v1,
               jnp.where(j == 2, v2, v3)))

    se_ref[...] = (sel4(Bm1, A0, B0, A1) + 3)[None]
    so_ref[...] = (sel4(A0, B0, A1, B1) + 3)[None]
    te_ref[...] = (sel4(A0, B0, A1, B1) + 3)[None]
    to_ref[...] = (sel4(B0, A1, B1, A2) + 3)[None]

    t = lax.broadcasted_iota(jnp.int32, (1, _NK), 1)
    p = t + j
    pi = jnp.where(p == 0, 0, ((p - 1) & 1) + 3)
    shift = jnp.maximum(j - 1, 0) // 2
    po = jnp.where(p == 0, 0, ((p - 1) >> 1) + 3 - shift)
    pi_ref[...] = pi[None]
    po_ref[...] = po[None]


def _vq_pack(js, y4, W_enc, W_pre, codebook):
    i32 = jnp.int32
    out_shapes = (
        jax.ShapeDtypeStruct((_B, 1, _HALF), i32),
        jax.ShapeDtypeStruct((_B, 1, _HALF), i32),
        jax.ShapeDtypeStruct((_B, 1, _HALF), i32),
        jax.ShapeDtypeStruct((_B, 1, _HALF), i32),
        jax.ShapeDtypeStruct((_B, 1, _NK), i32),
        jax.ShapeDtypeStruct((_B, 1, _NK), i32),
    )
    grid = (_B,)
    in_specs = [
        pl.BlockSpec((_B,), lambda k: (0,), memory_space=pltpu.SMEM),
        pl.BlockSpec(memory_space=pl.ANY),
        pl.BlockSpec((_D, _D), lambda k: (0, 0)),
        pl.BlockSpec((_D, _VQ), lambda k: (0, 0)),
        pl.BlockSpec((_NE, _VQ), lambda k: (0, 0)),
    ]
    out_specs = (
        pl.BlockSpec((1, 1, _HALF), lambda k: (k, 0, 0)),
        pl.BlockSpec((1, 1, _HALF), lambda k: (k, 0, 0)),
        pl.BlockSpec((1, 1, _HALF), lambda k: (k, 0, 0)),
        pl.BlockSpec((1, 1, _HALF), lambda k: (k, 0, 0)),
        pl.BlockSpec((1, 1, _NK), lambda k: (k, 0, 0)),
        pl.BlockSpec((1, 1, _NK), lambda k: (k, 0, 0)),
    )
    return pl.pallas_call(
        _vq_pack_body,
        grid=grid,
        in_specs=in_specs,
        out_specs=out_specs,
        out_shape=out_shapes,
        scratch_shapes=[pltpu.VMEM((2, 4, _NK, _DQ), jnp.float32),
                        pltpu.SemaphoreType.DMA],
    )(js, y4, W_enc, W_pre, codebook)


def kernel(x, edge_index, batch, faces, num_vertices, js, W_enc, W_pre, codebook):
    del batch, faces, num_vertices
    src = edge_index[0].astype(jnp.int32)
    dst = edge_index[1].astype(jnp.int32)
    js = js.astype(jnp.int32)

    x4 = jnp.concatenate([x[:, q * _DQ:(q + 1) * _DQ] for q in range(4)],
                         axis=0)
    src4 = jnp.stack([src + q * _N for q in range(4)]).reshape(
        4, _E // _CH, _CH)
    dst2 = dst.reshape(_E // _CH, _CH)
    y4 = _segsum_sc(src4, dst2, x4)

    se, so, te, to, pi, po = _vq_pack(js, y4, W_enc, W_pre, codebook)

    idxt = jax.dtypes.canonicalize_dtype(jnp.int64)
    S = jnp.stack([se.reshape(_B, _HALF), so.reshape(_B, _HALF)],
                  axis=-1).reshape(_B, _NK).astype(idxt)
    T = jnp.stack([te.reshape(_B, _HALF), to.reshape(_B, _HALF)],
                  axis=-1).reshape(_B, _NK).astype(idxt)
    PI = pi.reshape(_B, _NK).astype(idxt)
    PO = po.reshape(_B, _NK).astype(idxt)
    return S, T, PI, PO


# speedup vs baseline: 4.7857x; 1.0835x over previous
import functools

import jax
import jax.numpy as jnp
from jax import lax
from jax.experimental import pallas as pl
from jax.experimental.pallas import tpu as pltpu
from jax.experimental.pallas import tpu_sc as plsc

_N = 16384
_B = 8
_D = 192
_DH = 96
_VQ = 64
_NE = 1024
_NK = _N // _B
_HALF = _NK // 2

_E = 131072
_NSC = 16
_EPT = _E // _NSC
_CH = 128
_NCH = _EPT // _CH
_GRP = 4
_NGRP = _NCH // _GRP
_RPT = _N // _NSC


_DQ = _D // 4


def _segsum_body(src_hbm, dst_hbm, x4_hbm, out_hbm, srcv, dstv, rows, sem,
                 semb, semsc, shared):
    c = lax.axis_index("c")
    s = lax.axis_index("s")
    row0 = s * _RPT

    for t in range(2):
        q = 2 * c + t
        base = q * _N + row0
        for i in range(_RPT // _CH):
            pltpu.make_async_copy(x4_hbm.at[pl.ds(base + i * _CH, _CH)],
                                  shared.at[pl.ds(row0 + i * _CH, _CH)],
                                  semb).start()
        pltpu.sync_copy(src_hbm.at[q, pl.ds(s * _NCH, _NCH)], srcv)
        if t == 0:
            pltpu.sync_copy(dst_hbm.at[pl.ds(s * _NCH, _NCH)], dstv)

        def g_start(g, pbuf, sm):
            for b in range(_GRP):
                pltpu.make_async_copy(x4_hbm.at[srcv.at[g * _GRP + b]],
                                      rows.at[pbuf, b], sm).start()

        def g_wait(g, pbuf, sm):
            for b in range(_GRP):
                pltpu.make_async_copy(x4_hbm.at[srcv.at[g * _GRP + b]],
                                      rows.at[pbuf, b], sm).wait()

        def g_scatter(g, pbuf, sm):
            for b in range(_GRP):
                pltpu.make_async_copy(rows.at[pbuf, b],
                                      shared.at[dstv.at[g * _GRP + b]],
                                      sm).start(add=True)
            for b in range(_GRP):
                pltpu.make_async_copy(rows.at[pbuf, b],
                                      shared.at[dstv.at[g * _GRP + b]],
                                      sm).wait()

        g_start(0, 0, sem)
        for i in range(_RPT // _CH):
            pltpu.make_async_copy(x4_hbm.at[pl.ds(base + i * _CH, _CH)],
                                  shared.at[pl.ds(row0 + i * _CH, _CH)],
                                  semb).wait()
        plsc.subcore_barrier()

        def pair(i, carry):
            g0 = 2 * i
            g_start(g0 + 1, 1, semb)
            g_wait(g0, 0, sem)
            g_scatter(g0, 0, semsc)

            @pl.when(i < _NGRP // 2 - 1)
            def _():
                g_start(g0 + 2, 0, sem)

            g_wait(g0 + 1, 1, semb)
            g_scatter(g0 + 1, 1, semsc)
            return carry

        lax.fori_loop(0, _NGRP // 2, pair, 0)
        plsc.subcore_barrier()

        for i in range(_RPT // _CH):
            pltpu.make_async_copy(shared.at[pl.ds(row0 + i * _CH, _CH)],
                                  out_hbm.at[pl.ds(base + i * _CH, _CH)],
                                  semb).start()
        for i in range(_RPT // _CH):
            pltpu.make_async_copy(shared.at[pl.ds(row0 + i * _CH, _CH)],
                                  out_hbm.at[pl.ds(base + i * _CH, _CH)],
                                  semb).wait()


@functools.partial(
    pl.kernel,
    out_type=jax.ShapeDtypeStruct((4 * _N, _DQ), jnp.float32),
    mesh=plsc.VectorSubcoreMesh(core_axis_name="c", subcore_axis_name="s"),
    compiler_params=pltpu.CompilerParams(use_tc_tiling_on_sc=False),
    scratch_types=[
        pltpu.VMEM((_NCH, _CH), jnp.int32),
        pltpu.VMEM((_NCH, _CH), jnp.int32),
        pltpu.VMEM((2, _GRP, _CH, _DQ), jnp.float32),
        pltpu.SemaphoreType.DMA,
        pltpu.SemaphoreType.DMA,
        pltpu.SemaphoreType.DMA,
        pltpu.VMEM_SHARED((_N, _DQ), jnp.float32),
    ],
)
def _segsum_sc(src_hbm, dst_hbm, x4_hbm, out_hbm, srcv, dstv, rows, sem, semb,
               semsc, shared):
    _segsum_body(src_hbm, dst_hbm, x4_hbm, out_hbm, srcv, dstv, rows, sem,
                 semb, semsc, shared)


def _vq_pack_body(js_ref, y4_ref, wenc_ref, wpre_ref, cb_ref,
                  se_ref, so_ref, te_ref, to_ref, pi_ref, po_ref,
                  ybuf, ysem):
    k = pl.program_id(0)
    par = k % 2

    def start4(kk, pbuf):
        for q in range(4):
            pltpu.make_async_copy(
                y4_ref.at[pl.ds(q * _N + kk * _NK, _NK)],
                ybuf.at[pbuf, q], ysem).start()

    @pl.when(k == 0)
    def _():
        start4(k, 0)

    for q in range(4):
        pltpu.make_async_copy(y4_ref.at[pl.ds(0, _NK)], ybuf.at[0, q],
                              ysem).wait()

    @pl.when((k < _B - 1) & (par == 0))
    def _():
        start4(k + 1, 1)

    @pl.when((k < _B - 1) & (par == 1))
    def _():
        start4(k + 1, 0)

    dn = (((1,), (0,)), ((), ()))
    acc = None
    for q in range(4):
        yv = jnp.where(par == 0, ybuf[0, q], ybuf[1, q])
        wq = wenc_ref[q * _DQ:(q + 1) * _DQ, :]
        part = lax.dot_general(yv, wq, dn,
                               precision=lax.Precision.DEFAULT)
        acc = part if acc is None else acc + part
    h = jnp.maximum(acc, 0.0)
    e = lax.dot_general(h, wpre_ref[...], dn, precision=lax.Precision.DEFAULT)

    cb = cb_ref[...]
    cbn = jnp.sum(cb * cb, axis=1)[None, :]
    ii = lax.broadcasted_iota(jnp.int32, (_NK, _NE), 1)
    bf = jnp.bfloat16
    cb_hi = cb.astype(bf)
    cb_mid = (cb - cb_hi.astype(jnp.float32)).astype(bf)
    cb_lo = (cb - cb_hi.astype(jnp.float32)
             - cb_mid.astype(jnp.float32)).astype(bf)
    cb3 = jnp.concatenate([cb_hi, cb_mid, cb_lo], axis=1)

    def _level(r, need_residual):
        rn = jnp.sum(r * r, axis=1, keepdims=True)
        sc = lax.dot_general(r, cb, (((1,), (1,)), ((), ())),
                             precision=lax.Precision.DEFAULT)
        dists = rn - 2.0 * sc + cbn
        idx = jnp.argmin(dists, axis=1).astype(jnp.int32)
        if not need_residual:
            return idx, None
        oh = (ii == idx[:, None]).astype(bf)
        c3 = lax.dot_general(oh, cb3, dn, precision=lax.Precision.DEFAULT,
                             preferred_element_type=jnp.float32)
        c = ((c3[:, 0:_VQ] + c3[:, _VQ:2 * _VQ]) + c3[:, 2 * _VQ:3 * _VQ])
        return idx, r - c

    idx0, r1 = _level(e, True)
    idx1, _ = _level(r1, False)

    j = js_ref[pl.program_id(0)]
    A = idx0[None, :]
    Bv = idx1[None, :]
    A0 = A[:, 0:_HALF]
    A1 = A[:, 1:_HALF + 1]
    A2 = A[:, 2:_HALF + 2]
    B0 = Bv[:, 0:_HALF]
    B1 = Bv[:, 1:_HALF + 1]
    Bm1 = jnp.concatenate(
        [jnp.full((1, 1), -3, jnp.int32), Bv[:, 0:_HALF - 1]], axis=1)

    def sel4(v0, v1, v2, v3):
        return jnp.where(j == 0, v0,
               jnp.where(j == 1, v1,
               jnp.where(j == 2, v2, v3)))

    se_ref[...] = (sel4(Bm1, A0, B0, A1) + 3)[None]
    so_ref[...] = (sel4(A0, B0, A1, B1) + 3)[None]
    te_ref[...] = (sel4(A0, B0, A1, B1) + 3)[None]
    to_ref[...] = (sel4(B0, A1, B1, A2) + 3)[None]

    t = lax.broadcasted_iota(jnp.int32, (1, _NK), 1)
    p = t + j
    pi = jnp.where(p == 0, 0, ((p - 1) & 1) + 3)
    shift = jnp.maximum(j - 1, 0) // 2
    po = jnp.where(p == 0, 0, ((p - 1) >> 1) + 3 - shift)
    pi_ref[...] = pi[None]
    po_ref[...] = po[None]


def _vq_pack(js, y4, W_enc, W_pre, codebook):
    i32 = jnp.int32
    out_shapes = (
        jax.ShapeDtypeStruct((_B, 1, _HALF), i32),
        jax.ShapeDtypeStruct((_B, 1, _HALF), i32),
        jax.ShapeDtypeStruct((_B, 1, _HALF), i32),
        jax.ShapeDtypeStruct((_B, 1, _HALF), i32),
        jax.ShapeDtypeStruct((_B, 1, _NK), i32),
        jax.ShapeDtypeStruct((_B, 1, _NK), i32),
    )
    grid = (_B,)
    in_specs = [
        pl.BlockSpec((_B,), lambda k: (0,), memory_space=pltpu.SMEM),
        pl.BlockSpec(memory_space=pl.ANY),
        pl.BlockSpec((_D, _D), lambda k: (0, 0)),
        pl.BlockSpec((_D, _VQ), lambda k: (0, 0)),
        pl.BlockSpec((_NE, _VQ), lambda k: (0, 0)),
    ]
    out_specs = (
        pl.BlockSpec((1, 1, _HALF), lambda k: (k, 0, 0)),
        pl.BlockSpec((1, 1, _HALF), lambda k: (k, 0, 0)),
        pl.BlockSpec((1, 1, _HALF), lambda k: (k, 0, 0)),
        pl.BlockSpec((1, 1, _HALF), lambda k: (k, 0, 0)),
        pl.BlockSpec((1, 1, _NK), lambda k: (k, 0, 0)),
        pl.BlockSpec((1, 1, _NK), lambda k: (k, 0, 0)),
    )
    return pl.pallas_call(
        _vq_pack_body,
        grid=grid,
        in_specs=in_specs,
        out_specs=out_specs,
        out_shape=out_shapes,
        scratch_shapes=[pltpu.VMEM((2, 4, _NK, _DQ), jnp.float32),
                        pltpu.SemaphoreType.DMA],
    )(js, y4, W_enc, W_pre, codebook)


def kernel(x, edge_index, batch, faces, num_vertices, js, W_enc, W_pre, codebook):
    del batch, faces, num_vertices
    src = edge_index[0].astype(jnp.int32)
    dst = edge_index[1].astype(jnp.int32)
    js = js.astype(jnp.int32)

    x4 = jnp.concatenate([x[:, q * _DQ:(q + 1) * _DQ] for q in range(4)],
                         axis=0)
    src4 = jnp.stack([src + q * _N for q in range(4)]).reshape(
        4, _E // _CH, _CH)
    dst2 = dst.reshape(_E // _CH, _CH)
    y4 = _segsum_sc(src4, dst2, x4)

    se, so, te, to, pi, po = _vq_pack(js, y4, W_enc, W_pre, codebook)

    idxt = jax.dtypes.canonicalize_dtype(jnp.int64)
    S = jnp.stack([se.reshape(_B, _HALF), so.reshape(_B, _HALF)],
                  axis=-1).reshape(_B, _NK).astype(idxt)
    T = jnp.stack([te.reshape(_B, _HALF), to.reshape(_B, _HALF)],
                  axis=-1).reshape(_B, _NK).astype(idxt)
    PI = pi.reshape(_B, _NK).astype(idxt)
    PO = po.reshape(_B, _NK).astype(idxt)
    return S, T, PI, PO
